# Initial kernel scaffold; baseline (speedup 1.0000x reference)
import jax
import jax.numpy as jnp
from jax.experimental import pallas as pl


def kernel(x, edge_index, edge_attr, batch, Wm, bm, Wu, bu, a, gamma, beta, Wh1, bh1, ah, Wh2, bh2):
    raise NotImplementedError("write your pallas kernel here")


# speedup vs baseline: 2.4841x; 2.4841x over previous
import functools

import jax
import jax.numpy as jnp
from jax import lax
from jax.experimental import pallas as pl
from jax.experimental.pallas import tpu as pltpu
from jax.experimental.pallas import tpu_sc as plsc

N = 10000
E = 320000
D = 128
DE = 16
H = 128
L = 3
G = 64

NC = 2
NS = 16
NW = NC * NS
EPW = E // NW
K = 80
NCHUNK = EPW // K
NP = 10240
NPT = NP // NS
ZROWS = 128

BN = 400
NBLK = N // BN
BE = 2000

_f32 = jnp.float32
_HIGH = lax.Precision.HIGHEST


def _dot(a, b):
    return jnp.dot(a, b, precision=_HIGH, preferred_element_type=_f32)


def _edge_body(with_deg, *refs):
    if with_deg:
        (ps_hbm, pd_hbm, pe_hbm, src_hbm, dst_hbm,
         acc_out, deg_out,
         bufS, bufD, bufE, sidx, didx, zbuf, ones_b, zdeg,
         acc_sh, deg_sh, semS, semD, semE) = refs
    else:
        (ps_hbm, pd_hbm, pe_hbm, src_hbm, dst_hbm,
         acc_out,
         bufS, bufD, bufE, sidx, didx, zbuf,
         acc_sh, semS, semD, semE) = refs

    cid = lax.axis_index("c")
    sid = lax.axis_index("s")
    wid = sid * NC + cid

    zero16 = jnp.zeros((16,), _f32)

    def zrow(r, _):
        for v in range(H // 16):
            zbuf[r, pl.ds(v * 16, 16)] = zero16
        return 0
    lax.fori_loop(0, ZROWS, zrow, 0)
    for t in range(NPT // ZROWS):
        pltpu.sync_copy(zbuf, acc_sh.at[pl.ds(sid * NPT + t * ZROWS, ZROWS)])

    if with_deg:
        def zdrow(r, _):
            zdeg[pl.ds(r * 16, 16)] = zero16
            return 0
        lax.fori_loop(0, NPT // 16, zdrow, 0)
        pltpu.sync_copy(zdeg, deg_sh.at[pl.ds(sid * NPT, NPT)])

        one16 = jnp.ones((16,), _f32)

        def orow(r, _):
            ones_b[pl.ds(r * 16, 16)] = one16
            return 0
        lax.fori_loop(0, K // 16, orow, 0)

    plsc.subcore_barrier()

    def chunk(c, _):
        eoff = wid * EPW + c * K
        pltpu.sync_copy(src_hbm.at[pl.ds(eoff, K)], sidx)
        pltpu.sync_copy(dst_hbm.at[pl.ds(eoff, K)], didx)
        cS = pltpu.async_copy(ps_hbm.at[sidx], bufS, semS)
        cD = pltpu.async_copy(pd_hbm.at[didx], bufD, semD)
        cE = pltpu.async_copy(pe_hbm.at[pl.ds(eoff, K)], bufE, semE)
        cS.wait()
        cD.wait()
        cE.wait()

        def row(r, _):
            for v in range(H // 16):
                ix = (r, pl.ds(v * 16, 16))
                m = bufS[ix] + bufD[ix] + bufE[ix]
                bufS[ix] = jnp.maximum(m, 0.0)
            return 0
        lax.fori_loop(0, K, row, 0)

        pltpu.sync_copy(bufS, acc_sh.at[didx], add=True)
        if with_deg:
            pltpu.sync_copy(ones_b, deg_sh.at[didx], add=True)
        return 0
    lax.fori_loop(0, NCHUNK, chunk, 0)

    plsc.subcore_barrier()

    for t in range(NPT // ZROWS):
        off = sid * NPT + t * ZROWS
        pltpu.sync_copy(acc_sh.at[pl.ds(off, ZROWS)], zbuf)
        pltpu.sync_copy(zbuf, acc_out.at[cid, pl.ds(off, ZROWS)])
    if with_deg:
        pltpu.sync_copy(deg_sh.at[pl.ds(sid * NPT, NPT)], zdeg)
        pltpu.sync_copy(zdeg, deg_out.at[pl.ds(cid * NP + sid * NPT, NPT)])


def _make_edge_kernel(with_deg):
    out_type = [jax.ShapeDtypeStruct((NC, NP, H), _f32)]
    scratch = [
        pltpu.VMEM((K, H), _f32),
        pltpu.VMEM((K, H), _f32),
        pltpu.VMEM((K, H), _f32),
        pltpu.VMEM((K,), jnp.int32),
        pltpu.VMEM((K,), jnp.int32),
        pltpu.VMEM((ZROWS, H), _f32),
    ]
    if with_deg:
        out_type.append(jax.ShapeDtypeStruct((NC * NP,), _f32))
        scratch += [
            pltpu.VMEM((K,), _f32),
            pltpu.VMEM((NPT,), _f32),
        ]
    scratch += [pltpu.VMEM_SHARED((NP, H), _f32)]
    if with_deg:
        scratch += [pltpu.VMEM_SHARED((NP,), _f32)]
    scratch += [pltpu.SemaphoreType.DMA] * 3
    return pl.kernel(
        functools.partial(_edge_body, with_deg),
        out_type=out_type,
        mesh=plsc.VectorSubcoreMesh(
            core_axis_name="c", subcore_axis_name="s",
            num_cores=NC, num_subcores=NS),
        scratch_types=scratch,
        name="edge_deg" if with_deg else "edge",
    )


_edge_deg_kernel = _make_edge_kernel(True)
_edge_kernel = _make_edge_kernel(False)


def _pe_body(ea_ref, wm3_ref, pe_ref):
    pe_ref[0] = _dot(ea_ref[...], wm3_ref[0])


def _pe_call(edge_attr, Wm):
    return pl.pallas_call(
        _pe_body,
        grid=(L, E // BE),
        in_specs=[
            pl.BlockSpec((BE, DE), lambda l, e: (e, 0)),
            pl.BlockSpec((1, DE, H), lambda l, e: (l, 2 * H // DE, 0)),
        ],
        out_specs=pl.BlockSpec((1, BE, H), lambda l, e: (l, e, 0)),
        out_shape=jax.ShapeDtypeStruct((L, E, H), _f32),
    )(edge_attr, Wm)


def _proj_body(h_ref, w1_ref, w2_ref, bm_ref, ps_ref, pd_ref):
    h = h_ref[...]
    ps_ref[...] = _dot(h, w1_ref[...])
    pd_ref[...] = _dot(h, w2_ref[...]) + bm_ref[...]


def _proj_call(h, W1, W2, bm_i):
    return pl.pallas_call(
        _proj_body,
        grid=(NBLK,),
        in_specs=[
            pl.BlockSpec((BN, H), lambda i: (i, 0)),
            pl.BlockSpec((H, H), lambda i: (0, 0)),
            pl.BlockSpec((H, H), lambda i: (0, 0)),
            pl.BlockSpec((1, H), lambda i: (0, 0)),
        ],
        out_specs=[
            pl.BlockSpec((BN, H), lambda i: (i, 0)),
            pl.BlockSpec((BN, H), lambda i: (i, 0)),
        ],
        out_shape=[jax.ShapeDtypeStruct((N, H), _f32)] * 2,
    )(h, W1, W2, bm_i)


def _upd_body(h_ref, ap_ref, deg_ref, wu1_ref, wu2_ref, bu_ref, a_ref,
              h2_ref, sum_ref, ssq_ref):
    i = pl.program_id(0)
    agg = (ap_ref[0] + ap_ref[1]) / deg_ref[...]
    h2 = _dot(h_ref[...], wu1_ref[...]) + _dot(agg, wu2_ref[...]) + bu_ref[...]
    a = a_ref[0, 0]
    h2 = jnp.where(h2 > 0, h2, a * h2)
    h2_ref[...] = h2

    @pl.when(i == 0)
    def _():
        sum_ref[...] = jnp.zeros_like(sum_ref)
        ssq_ref[...] = jnp.zeros_like(ssq_ref)
    sum_ref[...] += jnp.sum(h2, axis=0, keepdims=True)
    ssq_ref[...] += jnp.sum(h2 * h2, axis=0, keepdims=True)


def _upd_call(h, acc_p, deg, Wu1, Wu2, bu_i, a_i):
    return pl.pallas_call(
        _upd_body,
        grid=(NBLK,),
        in_specs=[
            pl.BlockSpec((BN, H), lambda i: (i, 0)),
            pl.BlockSpec((NC, BN, H), lambda i: (0, i, 0)),
            pl.BlockSpec((BN, 1), lambda i: (i, 0)),
            pl.BlockSpec((H, H), lambda i: (0, 0)),
            pl.BlockSpec((H, H), lambda i: (0, 0)),
            pl.BlockSpec((1, H), lambda i: (0, 0)),
            pl.BlockSpec((1, 1), lambda i: (0, 0)),
        ],
        out_specs=[
            pl.BlockSpec((BN, H), lambda i: (i, 0)),
            pl.BlockSpec((1, H), lambda i: (0, 0)),
            pl.BlockSpec((1, H), lambda i: (0, 0)),
        ],
        out_shape=[
            jax.ShapeDtypeStruct((N, H), _f32),
            jax.ShapeDtypeStruct((1, H), _f32),
            jax.ShapeDtypeStruct((1, H), _f32),
        ],
    )(h, acc_p, deg, Wu1, Wu2, bu_i, a_i)


def _bn_scale_shift(sum_ref, ssq_ref, g_ref, b_ref):
    mu = sum_ref[...] / N
    var = ssq_ref[...] / N - mu * mu
    scale = lax.rsqrt(var + 1e-5) * g_ref[...]
    shift = b_ref[...] - mu * scale
    return scale, shift


def _norm_body(h2_ref, sum_ref, ssq_ref, g_ref, b_ref, w1_ref, w2_ref, bm_ref,
               h_ref, ps_ref, pd_ref):
    scale, shift = _bn_scale_shift(sum_ref, ssq_ref, g_ref, b_ref)
    h = h2_ref[...] * scale + shift
    h_ref[...] = h
    ps_ref[...] = _dot(h, w1_ref[...])
    pd_ref[...] = _dot(h, w2_ref[...]) + bm_ref[...]


def _norm_call(h2, ssum, ssq, gamma_i, beta_i, W1n, W2n, bm_n):
    return pl.pallas_call(
        _norm_body,
        grid=(NBLK,),
        in_specs=[
            pl.BlockSpec((BN, H), lambda i: (i, 0)),
            pl.BlockSpec((1, H), lambda i: (0, 0)),
            pl.BlockSpec((1, H), lambda i: (0, 0)),
            pl.BlockSpec((1, H), lambda i: (0, 0)),
            pl.BlockSpec((1, H), lambda i: (0, 0)),
            pl.BlockSpec((H, H), lambda i: (0, 0)),
            pl.BlockSpec((H, H), lambda i: (0, 0)),
            pl.BlockSpec((1, H), lambda i: (0, 0)),
        ],
        out_specs=[
            pl.BlockSpec((BN, H), lambda i: (i, 0)),
            pl.BlockSpec((BN, H), lambda i: (i, 0)),
            pl.BlockSpec((BN, H), lambda i: (i, 0)),
        ],
        out_shape=[jax.ShapeDtypeStruct((N, H), _f32)] * 3,
    )(h2, ssum, ssq, gamma_i, beta_i, W1n, W2n, bm_n)


def _final_body(h2_ref, sum_ref, ssq_ref, g_ref, b_ref, batch_ref,
                wh1_ref, bh1_ref, ah_ref, wh2_ref, bh2_ref,
                out_ref, pool_ref, cnt_ref):
    i = pl.program_id(0)
    scale, shift = _bn_scale_shift(sum_ref, ssq_ref, g_ref, b_ref)
    h = h2_ref[...] * scale + shift

    bb = batch_ref[0]
    gids = lax.broadcasted_iota(jnp.int32, (G, BN), 0)
    onehot = (gids == bb).astype(_f32)

    @pl.when(i == 0)
    def _():
        pool_ref[...] = jnp.zeros_like(pool_ref)
        cnt_ref[...] = jnp.zeros_like(cnt_ref)
    pool_ref[...] += _dot(onehot, h)
    cnt_ref[...] += _dot(onehot, jnp.ones_like(h))

    @pl.when(i == NBLK - 1)
    def _():
        pooled = pool_ref[...] / jnp.maximum(cnt_ref[...], 1.0)
        z = _dot(pooled, wh1_ref[...]) + bh1_ref[...]
        ah = ah_ref[0, 0]
        z = jnp.where(z > 0, z, ah * z)
        out_ref[...] = _dot(z, wh2_ref[...]) + bh2_ref[...]


def _final_call(h2, ssum, ssq, gamma_i, beta_i, batch3, Wh1, bh1, ah2, Wh2, bh2):
    return pl.pallas_call(
        _final_body,
        grid=(NBLK,),
        in_specs=[
            pl.BlockSpec((BN, H), lambda i: (i, 0)),
            pl.BlockSpec((1, H), lambda i: (0, 0)),
            pl.BlockSpec((1, H), lambda i: (0, 0)),
            pl.BlockSpec((1, H), lambda i: (0, 0)),
            pl.BlockSpec((1, H), lambda i: (0, 0)),
            pl.BlockSpec((1, 1, BN), lambda i: (i, 0, 0)),
            pl.BlockSpec((H, H), lambda i: (0, 0)),
            pl.BlockSpec((1, H), lambda i: (0, 0)),
            pl.BlockSpec((1, 1), lambda i: (0, 0)),
            pl.BlockSpec((H, 1), lambda i: (0, 0)),
            pl.BlockSpec((1, 1), lambda i: (0, 0)),
        ],
        out_specs=pl.BlockSpec((G, 1), lambda i: (0, 0)),
        out_shape=jax.ShapeDtypeStruct((G, 1), _f32),
        scratch_shapes=[
            pltpu.VMEM((G, H), _f32),
            pltpu.VMEM((G, H), _f32),
        ],
    )(h2, ssum, ssq, gamma_i, beta_i, batch3, Wh1, bh1, ah2, Wh2, bh2)


def kernel(x, edge_index, edge_attr, batch, Wm, bm, Wu, bu, a, gamma, beta,
           Wh1, bh1, ah, Wh2, bh2):
    src = edge_index[0].astype(jnp.int32)
    dst = edge_index[1].astype(jnp.int32)
    batch3 = batch.astype(jnp.int32).reshape(NBLK, 1, BN)

    pe_all = _pe_call(edge_attr, Wm)

    h = x
    ps, pd = _proj_call(h, Wm[0, :H], Wm[0, H:2 * H], bm[0:1])
    deg = None
    for i in range(L):
        if i == 0:
            acc_p, deg_flat = _edge_deg_kernel(ps, pd, pe_all[i], src, dst)
            deg = jnp.maximum(deg_flat[0:N] + deg_flat[NP:NP + N], 1.0)[:, None]
        else:
            (acc_p,) = _edge_kernel(ps, pd, pe_all[i], src, dst)
        h2, ssum, ssq = _upd_call(
            h, acc_p, deg, Wu[i, :H], Wu[i, H:], bu[i:i + 1],
            a[i].reshape(1, 1))
        if i < L - 1:
            h, ps, pd = _norm_call(
                h2, ssum, ssq, gamma[i:i + 1], beta[i:i + 1],
                Wm[i + 1, :H], Wm[i + 1, H:2 * H], bm[i + 1:i + 2])
        else:
            out = _final_call(
                h2, ssum, ssq, gamma[i:i + 1], beta[i:i + 1], batch3,
                Wh1, bh1.reshape(1, H), jnp.reshape(ah, (1, 1)),
                Wh2, bh2.reshape(1, 1))
    return out


# speedup vs baseline: 3.2755x; 1.3186x over previous
import functools

import jax
import jax.numpy as jnp
from jax import lax
from jax.experimental import pallas as pl
from jax.experimental.pallas import tpu as pltpu
from jax.experimental.pallas import tpu_sc as plsc

N = 10000
E = 320000
D = 128
DE = 16
H = 128
L = 3
G = 64

NC = 2
NS = 16
NW = NC * NS
EPW = E // NW
K = 80
NCHUNK = EPW // K
NP = 10240
NPT = NP // NS
ZROWS = 128

BN = 400
NBLK = N // BN
BE = 2000

_f32 = jnp.float32
_HIGH = lax.Precision.HIGHEST


def _dot(a, b):
    return jnp.dot(a, b, precision=_HIGH, preferred_element_type=_f32)


def _edge_body(with_deg, *refs):
    if with_deg:
        (ps_hbm, pd_hbm, pe_hbm, src_hbm, dst_hbm,
         acc_out, deg_out,
         bufS, bufD, bufE, sidx, didx, zbuf, ones_b, zdeg,
         acc_sh, deg_sh, semS, semD, semE) = refs
    else:
        (ps_hbm, pd_hbm, pe_hbm, src_hbm, dst_hbm,
         acc_out,
         bufS, bufD, bufE, sidx, didx, zbuf,
         acc_sh, semS, semD, semE) = refs

    cid = lax.axis_index("c")
    sid = lax.axis_index("s")
    wid = sid * NC + cid

    zero16 = jnp.zeros((16,), _f32)

    def zrow(r, _):
        for v in range(H // 16):
            zbuf[r, pl.ds(v * 16, 16)] = zero16
        return 0
    lax.fori_loop(0, ZROWS, zrow, 0)
    for t in range(NPT // ZROWS):
        pltpu.sync_copy(zbuf, acc_sh.at[pl.ds(sid * NPT + t * ZROWS, ZROWS)])

    if with_deg:
        def zdrow(r, _):
            zdeg[pl.ds(r * 16, 16)] = zero16
            return 0
        lax.fori_loop(0, NPT // 16, zdrow, 0)
        pltpu.sync_copy(zdeg, deg_sh.at[pl.ds(sid * NPT, NPT)])

        one16 = jnp.ones((16,), _f32)

        def orow(r, _):
            ones_b[pl.ds(r * 16, 16)] = one16
            return 0
        lax.fori_loop(0, K // 16, orow, 0)

    plsc.subcore_barrier()

    def chunk(c, _):
        eoff = wid * EPW + c * K
        pltpu.sync_copy(src_hbm.at[pl.ds(eoff, K)], sidx)
        pltpu.sync_copy(dst_hbm.at[pl.ds(eoff, K)], didx)
        cS = pltpu.async_copy(ps_hbm.at[sidx], bufS, semS)
        cD = pltpu.async_copy(pd_hbm.at[didx], bufD, semD)
        cE = pltpu.async_copy(pe_hbm.at[pl.ds(eoff, K)], bufE, semE)
        cS.wait()
        cD.wait()
        cE.wait()

        def row(r, _):
            for v in range(H // 16):
                ix = (r, pl.ds(v * 16, 16))
                m = bufS[ix] + bufD[ix] + bufE[ix]
                bufS[ix] = jnp.maximum(m, 0.0)
            return 0
        lax.fori_loop(0, K, row, 0)

        pltpu.sync_copy(bufS, acc_sh.at[didx], add=True)
        if with_deg:
            pltpu.sync_copy(ones_b, deg_sh.at[didx], add=True)
        return 0
    lax.fori_loop(0, NCHUNK, chunk, 0)

    plsc.subcore_barrier()

    for t in range(NPT // ZROWS):
        off = sid * NPT + t * ZROWS
        pltpu.sync_copy(acc_sh.at[pl.ds(off, ZROWS)], zbuf)
        pltpu.sync_copy(zbuf, acc_out.at[cid, pl.ds(off, ZROWS)])
    if with_deg:
        pltpu.sync_copy(deg_sh.at[pl.ds(sid * NPT, NPT)], zdeg)
        pltpu.sync_copy(zdeg, deg_out.at[pl.ds(cid * NP + sid * NPT, NPT)])


def _make_edge_kernel(with_deg):
    out_type = [jax.ShapeDtypeStruct((NC, NP, H), _f32)]
    scratch = [
        pltpu.VMEM((K, H), _f32),
        pltpu.VMEM((K, H), _f32),
        pltpu.VMEM((K, H), _f32),
        pltpu.VMEM((K,), jnp.int32),
        pltpu.VMEM((K,), jnp.int32),
        pltpu.VMEM((ZROWS, H), _f32),
    ]
    if with_deg:
        out_type.append(jax.ShapeDtypeStruct((NC * NP,), _f32))
        scratch += [
            pltpu.VMEM((K,), _f32),
            pltpu.VMEM((NPT,), _f32),
        ]
    scratch += [pltpu.VMEM_SHARED((NP, H), _f32)]
    if with_deg:
        scratch += [pltpu.VMEM_SHARED((NP,), _f32)]
    scratch += [pltpu.SemaphoreType.DMA] * 3
    return pl.kernel(
        functools.partial(_edge_body, with_deg),
        out_type=out_type,
        mesh=plsc.VectorSubcoreMesh(
            core_axis_name="c", subcore_axis_name="s",
            num_cores=NC, num_subcores=NS),
        scratch_types=scratch,
        name="edge_deg" if with_deg else "edge",
    )


_edge_deg_kernel = _make_edge_kernel(True)
_edge_kernel = _make_edge_kernel(False)


def _pe_body(ea_ref, wm3_ref, pe_ref):
    pe_ref[...] = _dot(ea_ref[...], wm3_ref[...])


def _pe_call(edge_attr, Wm3_i):
    return pl.pallas_call(
        _pe_body,
        grid=(E // BE,),
        in_specs=[
            pl.BlockSpec((BE, DE), lambda e: (e, 0)),
            pl.BlockSpec((DE, H), lambda e: (0, 0)),
        ],
        out_specs=pl.BlockSpec((BE, H), lambda e: (e, 0)),
        out_shape=jax.ShapeDtypeStruct((E, H), _f32),
        name="pe",
    )(edge_attr, Wm3_i)


def _proj_body(h_ref, w1_ref, w2_ref, bm_ref, ps_ref, pd_ref):
    h = h_ref[...]
    ps_ref[...] = _dot(h, w1_ref[...])
    pd_ref[...] = _dot(h, w2_ref[...]) + bm_ref[...]


def _proj_call(h, W1, W2, bm_i):
    return pl.pallas_call(
        _proj_body,
        grid=(NBLK,),
        in_specs=[
            pl.BlockSpec((BN, H), lambda i: (i, 0)),
            pl.BlockSpec((H, H), lambda i: (0, 0)),
            pl.BlockSpec((H, H), lambda i: (0, 0)),
            pl.BlockSpec((1, H), lambda i: (0, 0)),
        ],
        out_specs=[
            pl.BlockSpec((BN, H), lambda i: (i, 0)),
            pl.BlockSpec((BN, H), lambda i: (i, 0)),
        ],
        out_shape=[jax.ShapeDtypeStruct((N, H), _f32)] * 2,
    )(h, W1, W2, bm_i)


def _upd_body(h_ref, ap_ref, deg_ref, wu1_ref, wu2_ref, bu_ref, a_ref,
              h2_ref, sum_ref, ssq_ref):
    i = pl.program_id(0)
    agg = (ap_ref[0] + ap_ref[1]) / deg_ref[...]
    h2 = _dot(h_ref[...], wu1_ref[...]) + _dot(agg, wu2_ref[...]) + bu_ref[...]
    a = a_ref[0, 0]
    h2 = jnp.where(h2 > 0, h2, a * h2)
    h2_ref[...] = h2

    @pl.when(i == 0)
    def _():
        sum_ref[...] = jnp.zeros_like(sum_ref)
        ssq_ref[...] = jnp.zeros_like(ssq_ref)
    sum_ref[...] += jnp.sum(h2, axis=0, keepdims=True)
    ssq_ref[...] += jnp.sum(h2 * h2, axis=0, keepdims=True)


def _upd_call(h, acc_p, deg, Wu1, Wu2, bu_i, a_i):
    return pl.pallas_call(
        _upd_body,
        grid=(NBLK,),
        in_specs=[
            pl.BlockSpec((BN, H), lambda i: (i, 0)),
            pl.BlockSpec((NC, BN, H), lambda i: (0, i, 0)),
            pl.BlockSpec((BN, 1), lambda i: (i, 0)),
            pl.BlockSpec((H, H), lambda i: (0, 0)),
            pl.BlockSpec((H, H), lambda i: (0, 0)),
            pl.BlockSpec((1, H), lambda i: (0, 0)),
            pl.BlockSpec((1, 1), lambda i: (0, 0)),
        ],
        out_specs=[
            pl.BlockSpec((BN, H), lambda i: (i, 0)),
            pl.BlockSpec((1, H), lambda i: (0, 0)),
            pl.BlockSpec((1, H), lambda i: (0, 0)),
        ],
        out_shape=[
            jax.ShapeDtypeStruct((N, H), _f32),
            jax.ShapeDtypeStruct((1, H), _f32),
            jax.ShapeDtypeStruct((1, H), _f32),
        ],
    )(h, acc_p, deg, Wu1, Wu2, bu_i, a_i)


def _bn_scale_shift(sum_ref, ssq_ref, g_ref, b_ref):
    mu = sum_ref[...] / N
    var = ssq_ref[...] / N - mu * mu
    scale = lax.rsqrt(var + 1e-5) * g_ref[...]
    shift = b_ref[...] - mu * scale
    return scale, shift


def _norm_body(h2_ref, sum_ref, ssq_ref, g_ref, b_ref, w1_ref, w2_ref, bm_ref,
               h_ref, ps_ref, pd_ref):
    scale, shift = _bn_scale_shift(sum_ref, ssq_ref, g_ref, b_ref)
    h = h2_ref[...] * scale + shift
    h_ref[...] = h
    ps_ref[...] = _dot(h, w1_ref[...])
    pd_ref[...] = _dot(h, w2_ref[...]) + bm_ref[...]


def _norm_call(h2, ssum, ssq, gamma_i, beta_i, W1n, W2n, bm_n):
    return pl.pallas_call(
        _norm_body,
        grid=(NBLK,),
        in_specs=[
            pl.BlockSpec((BN, H), lambda i: (i, 0)),
            pl.BlockSpec((1, H), lambda i: (0, 0)),
            pl.BlockSpec((1, H), lambda i: (0, 0)),
            pl.BlockSpec((1, H), lambda i: (0, 0)),
            pl.BlockSpec((1, H), lambda i: (0, 0)),
            pl.BlockSpec((H, H), lambda i: (0, 0)),
            pl.BlockSpec((H, H), lambda i: (0, 0)),
            pl.BlockSpec((1, H), lambda i: (0, 0)),
        ],
        out_specs=[
            pl.BlockSpec((BN, H), lambda i: (i, 0)),
            pl.BlockSpec((BN, H), lambda i: (i, 0)),
            pl.BlockSpec((BN, H), lambda i: (i, 0)),
        ],
        out_shape=[jax.ShapeDtypeStruct((N, H), _f32)] * 3,
    )(h2, ssum, ssq, gamma_i, beta_i, W1n, W2n, bm_n)


def _final_body(h2_ref, sum_ref, ssq_ref, g_ref, b_ref, batch_ref,
                wh1_ref, bh1_ref, ah_ref, wh2_ref, bh2_ref,
                out_ref, pool_ref, cnt_ref):
    i = pl.program_id(0)
    scale, shift = _bn_scale_shift(sum_ref, ssq_ref, g_ref, b_ref)
    h = h2_ref[...] * scale + shift

    bb = batch_ref[0]
    gids = lax.broadcasted_iota(jnp.int32, (G, BN), 0)
    onehot = (gids == bb).astype(_f32)

    @pl.when(i == 0)
    def _():
        pool_ref[...] = jnp.zeros_like(pool_ref)
        cnt_ref[...] = jnp.zeros_like(cnt_ref)
    pool_ref[...] += _dot(onehot, h)
    cnt_ref[...] += _dot(onehot, jnp.ones_like(h))

    @pl.when(i == NBLK - 1)
    def _():
        pooled = pool_ref[...] / jnp.maximum(cnt_ref[...], 1.0)
        z = _dot(pooled, wh1_ref[...]) + bh1_ref[...]
        ah = ah_ref[0, 0]
        z = jnp.where(z > 0, z, ah * z)
        out_ref[...] = _dot(z, wh2_ref[...]) + bh2_ref[...]


def _final_call(h2, ssum, ssq, gamma_i, beta_i, batch3, Wh1, bh1, ah2, Wh2, bh2):
    return pl.pallas_call(
        _final_body,
        grid=(NBLK,),
        in_specs=[
            pl.BlockSpec((BN, H), lambda i: (i, 0)),
            pl.BlockSpec((1, H), lambda i: (0, 0)),
            pl.BlockSpec((1, H), lambda i: (0, 0)),
            pl.BlockSpec((1, H), lambda i: (0, 0)),
            pl.BlockSpec((1, H), lambda i: (0, 0)),
            pl.BlockSpec((1, 1, BN), lambda i: (i, 0, 0)),
            pl.BlockSpec((H, H), lambda i: (0, 0)),
            pl.BlockSpec((1, H), lambda i: (0, 0)),
            pl.BlockSpec((1, 1), lambda i: (0, 0)),
            pl.BlockSpec((H, 1), lambda i: (0, 0)),
            pl.BlockSpec((1, 1), lambda i: (0, 0)),
        ],
        out_specs=pl.BlockSpec((G, 1), lambda i: (0, 0)),
        out_shape=jax.ShapeDtypeStruct((G, 1), _f32),
        scratch_shapes=[
            pltpu.VMEM((G, H), _f32),
            pltpu.VMEM((G, H), _f32),
        ],
    )(h2, ssum, ssq, gamma_i, beta_i, batch3, Wh1, bh1, ah2, Wh2, bh2)


def kernel(x, edge_index, edge_attr, batch, Wm, bm, Wu, bu, a, gamma, beta,
           Wh1, bh1, ah, Wh2, bh2):
    src = edge_index[0].astype(jnp.int32)
    dst = edge_index[1].astype(jnp.int32)
    batch3 = batch.astype(jnp.int32).reshape(NBLK, 1, BN)

    pe = [_pe_call(edge_attr, Wm[i, 2 * H:]) for i in range(L)]

    h = x
    ps, pd = _proj_call(h, Wm[0, :H], Wm[0, H:2 * H], bm[0:1])
    deg = None
    for i in range(L):
        if i == 0:
            acc_p, deg_flat = _edge_deg_kernel(ps, pd, pe[i], src, dst)
            deg = jnp.maximum(deg_flat[0:N] + deg_flat[NP:NP + N], 1.0)[:, None]
        else:
            (acc_p,) = _edge_kernel(ps, pd, pe[i], src, dst)
        h2, ssum, ssq = _upd_call(
            h, acc_p, deg, Wu[i, :H], Wu[i, H:], bu[i:i + 1],
            a[i].reshape(1, 1))
        if i < L - 1:
            h, ps, pd = _norm_call(
                h2, ssum, ssq, gamma[i:i + 1], beta[i:i + 1],
                Wm[i + 1, :H], Wm[i + 1, H:2 * H], bm[i + 1:i + 2])
        else:
            out = _final_call(
                h2, ssum, ssq, gamma[i:i + 1], beta[i:i + 1], batch3,
                Wh1, bh1.reshape(1, H), jnp.reshape(ah, (1, 1)),
                Wh2, bh2.reshape(1, 1))
    return out


# speedup vs baseline: 3.8255x; 1.1679x over previous
import functools

import jax
import jax.numpy as jnp
from jax import lax
from jax.experimental import pallas as pl
from jax.experimental.pallas import tpu as pltpu
from jax.experimental.pallas import tpu_sc as plsc

N = 10000
E = 320000
D = 128
DE = 16
H = 128
L = 3
G = 64

NC = 2
NS = 16
NW = NC * NS
EPW = E // NW
K = 40
NCHUNK = EPW // K
NP = 10240
NPT = NP // NS
ZROWS = 64

BN = 400
NBLK = N // BN
BE = 2000

_f32 = jnp.float32
_HIGH = lax.Precision.HIGHEST


def _dot(a, b):
    return jnp.dot(a, b, precision=_HIGH, preferred_element_type=_f32)


def _edge_body(with_deg, *refs):
    if with_deg:
        (ps_hbm, pd_hbm, pe_hbm, src_hbm, dst_hbm,
         acc_out, deg_out,
         bufS0, bufD0, bufE0, sidx0, didx0,
         bufS1, bufD1, bufE1, sidx1, didx1,
         zbuf, ones_b, zdeg,
         acc_sh, deg_sh,
         semS0, semD0, semE0, semS1, semD1, semE1) = refs
    else:
        (ps_hbm, pd_hbm, pe_hbm, src_hbm, dst_hbm,
         acc_out,
         bufS0, bufD0, bufE0, sidx0, didx0,
         bufS1, bufD1, bufE1, sidx1, didx1,
         zbuf,
         acc_sh,
         semS0, semD0, semE0, semS1, semD1, semE1) = refs

    bufs = ((bufS0, bufD0, bufE0, sidx0, didx0, semS0, semD0, semE0),
            (bufS1, bufD1, bufE1, sidx1, didx1, semS1, semD1, semE1))

    cid = lax.axis_index("c")
    sid = lax.axis_index("s")
    wid = sid * NC + cid
    ebase = wid * EPW

    zero16 = jnp.zeros((16,), _f32)

    def zrow(r, _):
        for v in range(H // 16):
            zbuf[r, pl.ds(v * 16, 16)] = zero16
        return 0
    lax.fori_loop(0, ZROWS, zrow, 0)
    for t in range(NPT // ZROWS):
        pltpu.sync_copy(zbuf, acc_sh.at[pl.ds(sid * NPT + t * ZROWS, ZROWS)])

    if with_deg:
        def zdrow(r, _):
            zdeg[pl.ds(r * 16, 16)] = zero16
            return 0
        lax.fori_loop(0, NPT // 16, zdrow, 0)
        pltpu.sync_copy(zdeg, deg_sh.at[pl.ds(sid * NPT, NPT)])

        one16 = jnp.ones((16,), _f32)

        def orow(r, _):
            ones_b[pl.ds(r * 16, 16)] = one16
            return 0
        lax.fori_loop(0, K // 16, orow, 0)

    plsc.subcore_barrier()

    def issue(b, c):
        bufS, bufD, bufE, sidx, didx, semS, semD, semE = bufs[b]
        eoff = ebase + c * K
        pltpu.sync_copy(src_hbm.at[pl.ds(eoff, K)], sidx)
        pltpu.sync_copy(dst_hbm.at[pl.ds(eoff, K)], didx)
        pltpu.async_copy(ps_hbm.at[sidx], bufS, semS)
        pltpu.async_copy(pd_hbm.at[didx], bufD, semD)
        pltpu.async_copy(pe_hbm.at[pl.ds(eoff, K)], bufE, semE)

    def finish(b, c):
        bufS, bufD, bufE, sidx, didx, semS, semD, semE = bufs[b]
        eoff = ebase + c * K
        pltpu.make_async_copy(ps_hbm.at[sidx], bufS, semS).wait()
        pltpu.make_async_copy(pd_hbm.at[didx], bufD, semD).wait()
        pltpu.make_async_copy(pe_hbm.at[pl.ds(eoff, K)], bufE, semE).wait()

        def row(r, _):
            for v in range(H // 16):
                ix = (r, pl.ds(v * 16, 16))
                m = bufS[ix] + bufD[ix] + bufE[ix]
                bufS[ix] = jnp.maximum(m, 0.0)
            return 0
        lax.fori_loop(0, K, row, 0)

        pltpu.sync_copy(bufS, acc_sh.at[didx], add=True)
        if with_deg:
            pltpu.sync_copy(ones_b, deg_sh.at[didx], add=True)

    issue(0, 0)

    def body(i, _):
        c0 = 2 * i
        issue(1, c0 + 1)
        finish(0, c0)

        @pl.when(c0 + 2 < NCHUNK)
        def _():
            issue(0, c0 + 2)
        finish(1, c0 + 1)
        return 0
    lax.fori_loop(0, NCHUNK // 2, body, 0)
    if NCHUNK % 2 == 1:
        finish(0, NCHUNK - 1)

    plsc.subcore_barrier()

    for t in range(NPT // ZROWS):
        off = sid * NPT + t * ZROWS
        pltpu.sync_copy(acc_sh.at[pl.ds(off, ZROWS)], zbuf)
        pltpu.sync_copy(zbuf, acc_out.at[cid, pl.ds(off, ZROWS)])
    if with_deg:
        pltpu.sync_copy(deg_sh.at[pl.ds(sid * NPT, NPT)], zdeg)
        pltpu.sync_copy(zdeg, deg_out.at[pl.ds(cid * NP + sid * NPT, NPT)])


def _make_edge_kernel(with_deg):
    out_type = [jax.ShapeDtypeStruct((NC, NP, H), _f32)]
    scratch = [
        pltpu.VMEM((K, H), _f32),
        pltpu.VMEM((K, H), _f32),
        pltpu.VMEM((K, H), _f32),
        pltpu.VMEM((K,), jnp.int32),
        pltpu.VMEM((K,), jnp.int32),
        pltpu.VMEM((K, H), _f32),
        pltpu.VMEM((K, H), _f32),
        pltpu.VMEM((K, H), _f32),
        pltpu.VMEM((K,), jnp.int32),
        pltpu.VMEM((K,), jnp.int32),
        pltpu.VMEM((ZROWS, H), _f32),
    ]
    if with_deg:
        out_type.append(jax.ShapeDtypeStruct((NC * NP,), _f32))
        scratch += [
            pltpu.VMEM((K,), _f32),
            pltpu.VMEM((NPT,), _f32),
        ]
    scratch += [pltpu.VMEM_SHARED((NP, H), _f32)]
    if with_deg:
        scratch += [pltpu.VMEM_SHARED((NP,), _f32)]
    scratch += [pltpu.SemaphoreType.DMA] * 6
    return pl.kernel(
        functools.partial(_edge_body, with_deg),
        out_type=out_type,
        mesh=plsc.VectorSubcoreMesh(
            core_axis_name="c", subcore_axis_name="s",
            num_cores=NC, num_subcores=NS),
        scratch_types=scratch,
        name="edge_deg" if with_deg else "edge",
    )


_edge_deg_kernel = _make_edge_kernel(True)
_edge_kernel = _make_edge_kernel(False)


def _pe_body(ea_ref, wm3_ref, pe_ref):
    pe_ref[...] = _dot(ea_ref[...], wm3_ref[...])


def _pe_call(edge_attr, Wm3_i):
    return pl.pallas_call(
        _pe_body,
        grid=(E // BE,),
        in_specs=[
            pl.BlockSpec((BE, DE), lambda e: (e, 0)),
            pl.BlockSpec((DE, H), lambda e: (0, 0)),
        ],
        out_specs=pl.BlockSpec((BE, H), lambda e: (e, 0)),
        out_shape=jax.ShapeDtypeStruct((E, H), _f32),
        name="pe",
    )(edge_attr, Wm3_i)


def _proj_body(h_ref, w1_ref, w2_ref, bm_ref, ps_ref, pd_ref):
    h = h_ref[...]
    ps_ref[...] = _dot(h, w1_ref[...])
    pd_ref[...] = _dot(h, w2_ref[...]) + bm_ref[...]


def _proj_call(h, W1, W2, bm_i):
    return pl.pallas_call(
        _proj_body,
        grid=(NBLK,),
        in_specs=[
            pl.BlockSpec((BN, H), lambda i: (i, 0)),
            pl.BlockSpec((H, H), lambda i: (0, 0)),
            pl.BlockSpec((H, H), lambda i: (0, 0)),
            pl.BlockSpec((1, H), lambda i: (0, 0)),
        ],
        out_specs=[
            pl.BlockSpec((BN, H), lambda i: (i, 0)),
            pl.BlockSpec((BN, H), lambda i: (i, 0)),
        ],
        out_shape=[jax.ShapeDtypeStruct((N, H), _f32)] * 2,
    )(h, W1, W2, bm_i)


def _upd_body(h_ref, ap_ref, deg_ref, wu1_ref, wu2_ref, bu_ref, a_ref,
              h2_ref, sum_ref, ssq_ref):
    i = pl.program_id(0)
    agg = (ap_ref[0] + ap_ref[1]) / deg_ref[...]
    h2 = _dot(h_ref[...], wu1_ref[...]) + _dot(agg, wu2_ref[...]) + bu_ref[...]
    a = a_ref[0, 0]
    h2 = jnp.where(h2 > 0, h2, a * h2)
    h2_ref[...] = h2

    @pl.when(i == 0)
    def _():
        sum_ref[...] = jnp.zeros_like(sum_ref)
        ssq_ref[...] = jnp.zeros_like(ssq_ref)
    sum_ref[...] += jnp.sum(h2, axis=0, keepdims=True)
    ssq_ref[...] += jnp.sum(h2 * h2, axis=0, keepdims=True)


def _upd_call(h, acc_p, deg, Wu1, Wu2, bu_i, a_i):
    return pl.pallas_call(
        _upd_body,
        grid=(NBLK,),
        in_specs=[
            pl.BlockSpec((BN, H), lambda i: (i, 0)),
            pl.BlockSpec((NC, BN, H), lambda i: (0, i, 0)),
            pl.BlockSpec((BN, 1), lambda i: (i, 0)),
            pl.BlockSpec((H, H), lambda i: (0, 0)),
            pl.BlockSpec((H, H), lambda i: (0, 0)),
            pl.BlockSpec((1, H), lambda i: (0, 0)),
            pl.BlockSpec((1, 1), lambda i: (0, 0)),
        ],
        out_specs=[
            pl.BlockSpec((BN, H), lambda i: (i, 0)),
            pl.BlockSpec((1, H), lambda i: (0, 0)),
            pl.BlockSpec((1, H), lambda i: (0, 0)),
        ],
        out_shape=[
            jax.ShapeDtypeStruct((N, H), _f32),
            jax.ShapeDtypeStruct((1, H), _f32),
            jax.ShapeDtypeStruct((1, H), _f32),
        ],
    )(h, acc_p, deg, Wu1, Wu2, bu_i, a_i)


def _bn_scale_shift(sum_ref, ssq_ref, g_ref, b_ref):
    mu = sum_ref[...] / N
    var = ssq_ref[...] / N - mu * mu
    scale = lax.rsqrt(var + 1e-5) * g_ref[...]
    shift = b_ref[...] - mu * scale
    return scale, shift


def _norm_body(h2_ref, sum_ref, ssq_ref, g_ref, b_ref, w1_ref, w2_ref, bm_ref,
               h_ref, ps_ref, pd_ref):
    scale, shift = _bn_scale_shift(sum_ref, ssq_ref, g_ref, b_ref)
    h = h2_ref[...] * scale + shift
    h_ref[...] = h
    ps_ref[...] = _dot(h, w1_ref[...])
    pd_ref[...] = _dot(h, w2_ref[...]) + bm_ref[...]


def _norm_call(h2, ssum, ssq, gamma_i, beta_i, W1n, W2n, bm_n):
    return pl.pallas_call(
        _norm_body,
        grid=(NBLK,),
        in_specs=[
            pl.BlockSpec((BN, H), lambda i: (i, 0)),
            pl.BlockSpec((1, H), lambda i: (0, 0)),
            pl.BlockSpec((1, H), lambda i: (0, 0)),
            pl.BlockSpec((1, H), lambda i: (0, 0)),
            pl.BlockSpec((1, H), lambda i: (0, 0)),
            pl.BlockSpec((H, H), lambda i: (0, 0)),
            pl.BlockSpec((H, H), lambda i: (0, 0)),
            pl.BlockSpec((1, H), lambda i: (0, 0)),
        ],
        out_specs=[
            pl.BlockSpec((BN, H), lambda i: (i, 0)),
            pl.BlockSpec((BN, H), lambda i: (i, 0)),
            pl.BlockSpec((BN, H), lambda i: (i, 0)),
        ],
        out_shape=[jax.ShapeDtypeStruct((N, H), _f32)] * 3,
    )(h2, ssum, ssq, gamma_i, beta_i, W1n, W2n, bm_n)


def _final_body(h2_ref, sum_ref, ssq_ref, g_ref, b_ref, batch_ref,
                wh1_ref, bh1_ref, ah_ref, wh2_ref, bh2_ref,
                out_ref, pool_ref, cnt_ref):
    i = pl.program_id(0)
    scale, shift = _bn_scale_shift(sum_ref, ssq_ref, g_ref, b_ref)
    h = h2_ref[...] * scale + shift

    bb = batch_ref[0]
    gids = lax.broadcasted_iota(jnp.int32, (G, BN), 0)
    onehot = (gids == bb).astype(_f32)

    @pl.when(i == 0)
    def _():
        pool_ref[...] = jnp.zeros_like(pool_ref)
        cnt_ref[...] = jnp.zeros_like(cnt_ref)
    pool_ref[...] += _dot(onehot, h)
    cnt_ref[...] += _dot(onehot, jnp.ones_like(h))

    @pl.when(i == NBLK - 1)
    def _():
        pooled = pool_ref[...] / jnp.maximum(cnt_ref[...], 1.0)
        z = _dot(pooled, wh1_ref[...]) + bh1_ref[...]
        ah = ah_ref[0, 0]
        z = jnp.where(z > 0, z, ah * z)
        out_ref[...] = _dot(z, wh2_ref[...]) + bh2_ref[...]


def _final_call(h2, ssum, ssq, gamma_i, beta_i, batch3, Wh1, bh1, ah2, Wh2, bh2):
    return pl.pallas_call(
        _final_body,
        grid=(NBLK,),
        in_specs=[
            pl.BlockSpec((BN, H), lambda i: (i, 0)),
            pl.BlockSpec((1, H), lambda i: (0, 0)),
            pl.BlockSpec((1, H), lambda i: (0, 0)),
            pl.BlockSpec((1, H), lambda i: (0, 0)),
            pl.BlockSpec((1, H), lambda i: (0, 0)),
            pl.BlockSpec((1, 1, BN), lambda i: (i, 0, 0)),
            pl.BlockSpec((H, H), lambda i: (0, 0)),
            pl.BlockSpec((1, H), lambda i: (0, 0)),
            pl.BlockSpec((1, 1), lambda i: (0, 0)),
            pl.BlockSpec((H, 1), lambda i: (0, 0)),
            pl.BlockSpec((1, 1), lambda i: (0, 0)),
        ],
        out_specs=pl.BlockSpec((G, 1), lambda i: (0, 0)),
        out_shape=jax.ShapeDtypeStruct((G, 1), _f32),
        scratch_shapes=[
            pltpu.VMEM((G, H), _f32),
            pltpu.VMEM((G, H), _f32),
        ],
    )(h2, ssum, ssq, gamma_i, beta_i, batch3, Wh1, bh1, ah2, Wh2, bh2)


def kernel(x, edge_index, edge_attr, batch, Wm, bm, Wu, bu, a, gamma, beta,
           Wh1, bh1, ah, Wh2, bh2):
    src = edge_index[0].astype(jnp.int32)
    dst = edge_index[1].astype(jnp.int32)
    batch3 = batch.astype(jnp.int32).reshape(NBLK, 1, BN)

    pe = [_pe_call(edge_attr, Wm[i, 2 * H:]) for i in range(L)]

    h = x
    ps, pd = _proj_call(h, Wm[0, :H], Wm[0, H:2 * H], bm[0:1])
    deg = None
    for i in range(L):
        if i == 0:
            acc_p, deg_flat = _edge_deg_kernel(ps, pd, pe[i], src, dst)
            deg = jnp.maximum(deg_flat[0:N] + deg_flat[NP:NP + N], 1.0)[:, None]
        else:
            (acc_p,) = _edge_kernel(ps, pd, pe[i], src, dst)
        h2, ssum, ssq = _upd_call(
            h, acc_p, deg, Wu[i, :H], Wu[i, H:], bu[i:i + 1],
            a[i].reshape(1, 1))
        if i < L - 1:
            h, ps, pd = _norm_call(
                h2, ssum, ssq, gamma[i:i + 1], beta[i:i + 1],
                Wm[i + 1, :H], Wm[i + 1, H:2 * H], bm[i + 1:i + 2])
        else:
            out = _final_call(
                h2, ssum, ssq, gamma[i:i + 1], beta[i:i + 1], batch3,
                Wh1, bh1.reshape(1, H), jnp.reshape(ah, (1, 1)),
                Wh2, bh2.reshape(1, 1))
    return out


# speedup vs baseline: 4.4533x; 1.1641x over previous
import jax
import jax.numpy as jnp
from jax import lax
from jax.experimental import pallas as pl
from jax.experimental.pallas import tpu as pltpu
from jax.experimental.pallas import tpu_sc as plsc

N = 10000
E = 320000
D = 128
DE = 16
H = 128
L = 3
G = 64

NC = 2
NS = 16
NW = NC * NS
EPW = E // NW
K = 64
NFULL = EPW // K
TK = EPW - NFULL * K
NP = 10112
NPT = NP // NS

KD = 128
NFULLD = EPW // KD
TKD = EPW - NFULLD * KD

BN = 400
NBLK = N // BN
BE = 2000

_f32 = jnp.float32
_HIGH = lax.Precision.HIGHEST


def _dot(a, b):
    return jnp.dot(a, b, precision=_HIGH, preferred_element_type=_f32)


def _edge_body(ps_hbm, pd_hbm, pe_hbm, src_hbm, dst_hbm,
               acc_out,
               bufS0, bufD0, bufE0, sidx0, didx0,
               bufS1, bufD1, bufE1, sidx1, didx1,
               tsidx, tdidx,
               acc_sh,
               semS0, semD0, semE0, semS1, semD1, semE1):
    bufs = ((bufS0, bufD0, bufE0, sidx0, didx0, semS0, semD0, semE0),
            (bufS1, bufD1, bufE1, sidx1, didx1, semS1, semD1, semE1))

    cid = lax.axis_index("c")
    sid = lax.axis_index("s")
    wid = sid * NC + cid
    ebase = wid * EPW

    zero16 = jnp.zeros((16,), _f32)

    def zrow(r, _):
        for v in range(H // 16):
            bufS0[r, pl.ds(v * 16, 16)] = zero16
        return 0
    lax.fori_loop(0, K, zrow, 0)
    for t in range(NPT // K):
        pltpu.sync_copy(bufS0, acc_sh.at[pl.ds(sid * NPT + t * K, K)])
    rem = NPT - (NPT // K) * K
    if rem:
        pltpu.sync_copy(bufS0.at[pl.ds(0, rem)],
                        acc_sh.at[pl.ds(sid * NPT + NPT - rem, rem)])

    plsc.subcore_barrier()

    def issue(b, c):
        bufS, bufD, bufE, sidx, didx, semS, semD, semE = bufs[b]
        eoff = ebase + c * K
        pltpu.sync_copy(src_hbm.at[pl.ds(eoff, K)], sidx)
        pltpu.sync_copy(dst_hbm.at[pl.ds(eoff, K)], didx)
        pltpu.async_copy(ps_hbm.at[sidx], bufS, semS)
        pltpu.async_copy(pd_hbm.at[didx], bufD, semD)
        pltpu.async_copy(pe_hbm.at[pl.ds(eoff, K)], bufE, semE)

    def finish(b, c):
        bufS, bufD, bufE, sidx, didx, semS, semD, semE = bufs[b]
        eoff = ebase + c * K
        pltpu.make_async_copy(ps_hbm.at[sidx], bufS, semS).wait()
        pltpu.make_async_copy(pd_hbm.at[didx], bufD, semD).wait()
        pltpu.make_async_copy(pe_hbm.at[pl.ds(eoff, K)], bufE, semE).wait()

        def row(r, _):
            for v in range(H // 16):
                ix = (r, pl.ds(v * 16, 16))
                m = bufS[ix] + bufD[ix] + bufE[ix]
                bufS[ix] = jnp.maximum(m, 0.0)
            return 0
        lax.fori_loop(0, K, row, 0)

        pltpu.sync_copy(bufS, acc_sh.at[didx], add=True)

    issue(0, 0)

    def body(i, _):
        c0 = 2 * i
        issue(1, c0 + 1)
        finish(0, c0)

        @pl.when(c0 + 2 < NFULL)
        def _():
            issue(0, c0 + 2)
        finish(1, c0 + 1)
        return 0
    lax.fori_loop(0, NFULL // 2, body, 0)

    toff = ebase + NFULL * K
    pltpu.sync_copy(src_hbm.at[pl.ds(toff, TK)], tsidx)
    pltpu.sync_copy(dst_hbm.at[pl.ds(toff, TK)], tdidx)
    cS = pltpu.async_copy(ps_hbm.at[tsidx], bufS0.at[pl.ds(0, TK)], semS0)
    cD = pltpu.async_copy(pd_hbm.at[tdidx], bufD0.at[pl.ds(0, TK)], semD0)
    cE = pltpu.async_copy(pe_hbm.at[pl.ds(toff, TK)], bufE0.at[pl.ds(0, TK)],
                          semE0)
    cS.wait()
    cD.wait()
    cE.wait()

    def trow(r, _):
        for v in range(H // 16):
            ix = (r, pl.ds(v * 16, 16))
            m = bufS0[ix] + bufD0[ix] + bufE0[ix]
            bufS0[ix] = jnp.maximum(m, 0.0)
        return 0
    lax.fori_loop(0, TK, trow, 0)
    pltpu.sync_copy(bufS0.at[pl.ds(0, TK)], acc_sh.at[tdidx], add=True)

    plsc.subcore_barrier()

    for t in range(NPT // K):
        off = sid * NPT + t * K
        pltpu.sync_copy(acc_sh.at[pl.ds(off, K)], bufS0)
        pltpu.sync_copy(bufS0, acc_out.at[cid, pl.ds(off, K)])
    if rem:
        off = sid * NPT + NPT - rem
        pltpu.sync_copy(acc_sh.at[pl.ds(off, rem)], bufS0.at[pl.ds(0, rem)])
        pltpu.sync_copy(bufS0.at[pl.ds(0, rem)],
                        acc_out.at[cid, pl.ds(off, rem)])


_edge_kernel = pl.kernel(
    _edge_body,
    out_type=[jax.ShapeDtypeStruct((NC, NP, H), _f32)],
    mesh=plsc.VectorSubcoreMesh(
        core_axis_name="c", subcore_axis_name="s",
        num_cores=NC, num_subcores=NS),
    scratch_types=[
        pltpu.VMEM((K, H), _f32),
        pltpu.VMEM((K, H), _f32),
        pltpu.VMEM((K, H), _f32),
        pltpu.VMEM((K,), jnp.int32),
        pltpu.VMEM((K,), jnp.int32),
        pltpu.VMEM((K, H), _f32),
        pltpu.VMEM((K, H), _f32),
        pltpu.VMEM((K, H), _f32),
        pltpu.VMEM((K,), jnp.int32),
        pltpu.VMEM((K,), jnp.int32),
        pltpu.VMEM((TK,), jnp.int32),
        pltpu.VMEM((TK,), jnp.int32),
        pltpu.VMEM_SHARED((NP, H), _f32),
    ] + [pltpu.SemaphoreType.DMA] * 6,
    name="edge",
)


def _deg_body(dst_hbm, deg_out, ones_b, didx, tdidx, zdeg, deg_sh):
    cid = lax.axis_index("c")
    sid = lax.axis_index("s")
    wid = sid * NC + cid
    ebase = wid * EPW

    zero16 = jnp.zeros((16,), _f32)
    one16 = jnp.ones((16,), _f32)

    def zrow(r, _):
        zdeg[pl.ds(r * 16, 16)] = zero16
        return 0
    lax.fori_loop(0, 640 // 16, zrow, 0)
    pltpu.sync_copy(zdeg.at[pl.ds(0, NPT)], deg_sh.at[pl.ds(sid * NPT, NPT)])

    def orow(r, _):
        ones_b[pl.ds(r * 16, 16)] = one16
        return 0
    lax.fori_loop(0, KD // 16, orow, 0)

    plsc.subcore_barrier()

    def chunk(c, _):
        pltpu.sync_copy(dst_hbm.at[pl.ds(ebase + c * KD, KD)], didx)
        pltpu.sync_copy(ones_b, deg_sh.at[didx], add=True)
        return 0
    lax.fori_loop(0, NFULLD, chunk, 0)

    pltpu.sync_copy(dst_hbm.at[pl.ds(ebase + NFULLD * KD, TKD)], tdidx)
    pltpu.sync_copy(ones_b.at[pl.ds(0, TKD)], deg_sh.at[tdidx], add=True)

    plsc.subcore_barrier()

    pltpu.sync_copy(deg_sh.at[pl.ds(sid * NPT, NPT)], zdeg.at[pl.ds(0, NPT)])
    pltpu.sync_copy(zdeg.at[pl.ds(0, NPT)],
                    deg_out.at[pl.ds(cid * NP + sid * NPT, NPT)])


_deg_kernel = pl.kernel(
    _deg_body,
    out_type=[jax.ShapeDtypeStruct((NC * NP,), _f32)],
    mesh=plsc.VectorSubcoreMesh(
        core_axis_name="c", subcore_axis_name="s",
        num_cores=NC, num_subcores=NS),
    scratch_types=[
        pltpu.VMEM((KD,), _f32),
        pltpu.VMEM((KD,), jnp.int32),
        pltpu.VMEM((TKD,), jnp.int32),
        pltpu.VMEM((640,), _f32),
        pltpu.VMEM_SHARED((NP,), _f32),
    ],
    name="deg",
)


def _pe_body(ea_ref, wm3_ref, pe_ref):
    pe_ref[...] = _dot(ea_ref[...], wm3_ref[...])


def _pe_call(edge_attr, Wm3_i):
    return pl.pallas_call(
        _pe_body,
        grid=(E // BE,),
        in_specs=[
            pl.BlockSpec((BE, DE), lambda e: (e, 0)),
            pl.BlockSpec((DE, H), lambda e: (0, 0)),
        ],
        out_specs=pl.BlockSpec((BE, H), lambda e: (e, 0)),
        out_shape=jax.ShapeDtypeStruct((E, H), _f32),
        name="pe",
    )(edge_attr, Wm3_i)


def _proj_body(h_ref, w1_ref, w2_ref, bm_ref, ps_ref, pd_ref):
    h = h_ref[...]
    ps_ref[...] = _dot(h, w1_ref[...])
    pd_ref[...] = _dot(h, w2_ref[...]) + bm_ref[...]


def _proj_call(h, W1, W2, bm_i):
    return pl.pallas_call(
        _proj_body,
        grid=(NBLK,),
        in_specs=[
            pl.BlockSpec((BN, H), lambda i: (i, 0)),
            pl.BlockSpec((H, H), lambda i: (0, 0)),
            pl.BlockSpec((H, H), lambda i: (0, 0)),
            pl.BlockSpec((1, H), lambda i: (0, 0)),
        ],
        out_specs=[
            pl.BlockSpec((BN, H), lambda i: (i, 0)),
            pl.BlockSpec((BN, H), lambda i: (i, 0)),
        ],
        out_shape=[jax.ShapeDtypeStruct((N, H), _f32)] * 2,
        name="proj",
    )(h, W1, W2, bm_i)


def _upd_body(h_ref, ap_ref, deg_ref, wu1_ref, wu2_ref, bu_ref, a_ref,
              h2_ref, sum_ref, ssq_ref):
    i = pl.program_id(0)
    agg = (ap_ref[0] + ap_ref[1]) / deg_ref[...]
    h2 = _dot(h_ref[...], wu1_ref[...]) + _dot(agg, wu2_ref[...]) + bu_ref[...]
    a = a_ref[0, 0]
    h2 = jnp.where(h2 > 0, h2, a * h2)
    h2_ref[...] = h2

    @pl.when(i == 0)
    def _():
        sum_ref[...] = jnp.zeros_like(sum_ref)
        ssq_ref[...] = jnp.zeros_like(ssq_ref)
    sum_ref[...] += jnp.sum(h2, axis=0, keepdims=True)
    ssq_ref[...] += jnp.sum(h2 * h2, axis=0, keepdims=True)


def _upd_call(h, acc_p, deg, Wu1, Wu2, bu_i, a_i):
    return pl.pallas_call(
        _upd_body,
        grid=(NBLK,),
        in_specs=[
            pl.BlockSpec((BN, H), lambda i: (i, 0)),
            pl.BlockSpec((NC, BN, H), lambda i: (0, i, 0)),
            pl.BlockSpec((BN, 1), lambda i: (i, 0)),
            pl.BlockSpec((H, H), lambda i: (0, 0)),
            pl.BlockSpec((H, H), lambda i: (0, 0)),
            pl.BlockSpec((1, H), lambda i: (0, 0)),
            pl.BlockSpec((1, 1), lambda i: (0, 0)),
        ],
        out_specs=[
            pl.BlockSpec((BN, H), lambda i: (i, 0)),
            pl.BlockSpec((1, H), lambda i: (0, 0)),
            pl.BlockSpec((1, H), lambda i: (0, 0)),
        ],
        out_shape=[
            jax.ShapeDtypeStruct((N, H), _f32),
            jax.ShapeDtypeStruct((1, H), _f32),
            jax.ShapeDtypeStruct((1, H), _f32),
        ],
        name="upd",
    )(h, acc_p, deg, Wu1, Wu2, bu_i, a_i)


def _bn_scale_shift(sum_ref, ssq_ref, g_ref, b_ref):
    mu = sum_ref[...] / N
    var = ssq_ref[...] / N - mu * mu
    scale = lax.rsqrt(var + 1e-5) * g_ref[...]
    shift = b_ref[...] - mu * scale
    return scale, shift


def _norm_body(h2_ref, sum_ref, ssq_ref, g_ref, b_ref, w1_ref, w2_ref, bm_ref,
               h_ref, ps_ref, pd_ref):
    scale, shift = _bn_scale_shift(sum_ref, ssq_ref, g_ref, b_ref)
    h = h2_ref[...] * scale + shift
    h_ref[...] = h
    ps_ref[...] = _dot(h, w1_ref[...])
    pd_ref[...] = _dot(h, w2_ref[...]) + bm_ref[...]


def _norm_call(h2, ssum, ssq, gamma_i, beta_i, W1n, W2n, bm_n):
    return pl.pallas_call(
        _norm_body,
        grid=(NBLK,),
        in_specs=[
            pl.BlockSpec((BN, H), lambda i: (i, 0)),
            pl.BlockSpec((1, H), lambda i: (0, 0)),
            pl.BlockSpec((1, H), lambda i: (0, 0)),
            pl.BlockSpec((1, H), lambda i: (0, 0)),
            pl.BlockSpec((1, H), lambda i: (0, 0)),
            pl.BlockSpec((H, H), lambda i: (0, 0)),
            pl.BlockSpec((H, H), lambda i: (0, 0)),
            pl.BlockSpec((1, H), lambda i: (0, 0)),
        ],
        out_specs=[
            pl.BlockSpec((BN, H), lambda i: (i, 0)),
            pl.BlockSpec((BN, H), lambda i: (i, 0)),
            pl.BlockSpec((BN, H), lambda i: (i, 0)),
        ],
        out_shape=[jax.ShapeDtypeStruct((N, H), _f32)] * 3,
        name="norm",
    )(h2, ssum, ssq, gamma_i, beta_i, W1n, W2n, bm_n)


def _final_body(h2_ref, sum_ref, ssq_ref, g_ref, b_ref, batch_ref,
                wh1_ref, bh1_ref, ah_ref, wh2_ref, bh2_ref,
                out_ref, pool_ref, cnt_ref):
    i = pl.program_id(0)
    scale, shift = _bn_scale_shift(sum_ref, ssq_ref, g_ref, b_ref)
    h = h2_ref[...] * scale + shift

    bb = batch_ref[0]
    gids = lax.broadcasted_iota(jnp.int32, (G, BN), 0)
    onehot = (gids == bb).astype(_f32)

    @pl.when(i == 0)
    def _():
        pool_ref[...] = jnp.zeros_like(pool_ref)
        cnt_ref[...] = jnp.zeros_like(cnt_ref)
    pool_ref[...] += _dot(onehot, h)
    cnt_ref[...] += _dot(onehot, jnp.ones_like(h))

    @pl.when(i == NBLK - 1)
    def _():
        pooled = pool_ref[...] / jnp.maximum(cnt_ref[...], 1.0)
        z = _dot(pooled, wh1_ref[...]) + bh1_ref[...]
        ah = ah_ref[0, 0]
        z = jnp.where(z > 0, z, ah * z)
        out_ref[...] = _dot(z, wh2_ref[...]) + bh2_ref[...]


def _final_call(h2, ssum, ssq, gamma_i, beta_i, batch3, Wh1, bh1, ah2, Wh2, bh2):
    return pl.pallas_call(
        _final_body,
        grid=(NBLK,),
        in_specs=[
            pl.BlockSpec((BN, H), lambda i: (i, 0)),
            pl.BlockSpec((1, H), lambda i: (0, 0)),
            pl.BlockSpec((1, H), lambda i: (0, 0)),
            pl.BlockSpec((1, H), lambda i: (0, 0)),
            pl.BlockSpec((1, H), lambda i: (0, 0)),
            pl.BlockSpec((1, 1, BN), lambda i: (i, 0, 0)),
            pl.BlockSpec((H, H), lambda i: (0, 0)),
            pl.BlockSpec((1, H), lambda i: (0, 0)),
            pl.BlockSpec((1, 1), lambda i: (0, 0)),
            pl.BlockSpec((H, 1), lambda i: (0, 0)),
            pl.BlockSpec((1, 1), lambda i: (0, 0)),
        ],
        out_specs=pl.BlockSpec((G, 1), lambda i: (0, 0)),
        out_shape=jax.ShapeDtypeStruct((G, 1), _f32),
        scratch_shapes=[
            pltpu.VMEM((G, H), _f32),
            pltpu.VMEM((G, H), _f32),
        ],
        name="head",
    )(h2, ssum, ssq, gamma_i, beta_i, batch3, Wh1, bh1, ah2, Wh2, bh2)


def kernel(x, edge_index, edge_attr, batch, Wm, bm, Wu, bu, a, gamma, beta,
           Wh1, bh1, ah, Wh2, bh2):
    src = edge_index[0].astype(jnp.int32)
    dst = edge_index[1].astype(jnp.int32)
    batch3 = batch.astype(jnp.int32).reshape(NBLK, 1, BN)

    pe = [_pe_call(edge_attr, Wm[i, 2 * H:]) for i in range(L)]
    (deg_flat,) = _deg_kernel(dst)
    deg = jnp.maximum(deg_flat[0:N] + deg_flat[NP:NP + N], 1.0)[:, None]

    h = x
    ps, pd = _proj_call(h, Wm[0, :H], Wm[0, H:2 * H], bm[0:1])
    for i in range(L):
        (acc_p,) = _edge_kernel(ps, pd, pe[i], src, dst)
        h2, ssum, ssq = _upd_call(
            h, acc_p, deg, Wu[i, :H], Wu[i, H:], bu[i:i + 1],
            a[i].reshape(1, 1))
        if i < L - 1:
            h, ps, pd = _norm_call(
                h2, ssum, ssq, gamma[i:i + 1], beta[i:i + 1],
                Wm[i + 1, :H], Wm[i + 1, H:2 * H], bm[i + 1:i + 2])
        else:
            out = _final_call(
                h2, ssum, ssq, gamma[i:i + 1], beta[i:i + 1], batch3,
                Wh1, bh1.reshape(1, H), jnp.reshape(ah, (1, 1)),
                Wh2, bh2.reshape(1, 1))
    return out


# speedup vs baseline: 4.4557x; 1.0005x over previous
import jax
import jax.numpy as jnp
from jax import lax
from jax.experimental import pallas as pl
from jax.experimental.pallas import tpu as pltpu
from jax.experimental.pallas import tpu_sc as plsc

N = 10000
E = 320000
D = 128
DE = 16
H = 128
L = 3
G = 64

NC = 2
NS = 16
NW = NC * NS
EPW = E // NW
K = 64
NFULL = EPW // K
TK = EPW - NFULL * K
NP = 10112
NPT = NP // NS

KD = 128
NFULLD = EPW // KD
TKD = EPW - NFULLD * KD

BN = 400
NBLK = N // BN
BE = 2000
HP = H // 2

_f32 = jnp.float32
_HIGH = lax.Precision.HIGHEST

def _dot(a, b):
    return jnp.dot(a, b, precision=_HIGH, preferred_element_type=_f32)


def _edge_body(ps_hbm, pd_hbm, pe_hbm, src_hbm, dst_hbm,
               acc_out,
               bufS0, bufD0, bufE0, sidx0, didx0,
               bufS1, bufD1, bufE1, sidx1, didx1,
               tsidx, tdidx,
               acc_sh,
               semS0, semD0, semE0, semW0,
               semS1, semD1, semE1, semW1):
    bufs = ((bufS0, bufD0, bufE0, sidx0, didx0, semS0, semD0, semE0, semW0),
            (bufS1, bufD1, bufE1, sidx1, didx1, semS1, semD1, semE1, semW1))

    cid = lax.axis_index("c")
    sid = lax.axis_index("s")
    wid = sid * NC + cid
    ebase = wid * EPW

    zero16 = jnp.zeros((16,), _f32)

    def zrow(r, _):
        for v in range(H // 16):
            bufS0[r, pl.ds(v * 16, 16)] = zero16
        return 0
    lax.fori_loop(0, K, zrow, 0)
    for t in range(NPT // K):
        pltpu.sync_copy(bufS0, acc_sh.at[pl.ds(sid * NPT + t * K, K)])
    rem = NPT - (NPT // K) * K
    if rem:
        pltpu.sync_copy(bufS0.at[pl.ds(0, rem)],
                        acc_sh.at[pl.ds(sid * NPT + NPT - rem, rem)])

    plsc.subcore_barrier()

    def issue(b, c):
        bufS, bufD, bufE, sidx, didx, semS, semD, semE, semW = bufs[b]
        eoff = ebase + c * K
        @pl.when(c >= 2)
        def _():
            pltpu.make_async_copy(bufS, acc_sh.at[didx], semW).wait()
        pltpu.sync_copy(src_hbm.at[pl.ds(eoff, K)], sidx)
        pltpu.sync_copy(dst_hbm.at[pl.ds(eoff, K)], didx)
        pltpu.async_copy(ps_hbm.at[sidx], bufS, semS)
        pltpu.async_copy(pd_hbm.at[didx], bufD, semD)
        pltpu.async_copy(pe_hbm.at[pl.ds(eoff, K)], bufE, semE)

    def finish(b, c):
        bufS, bufD, bufE, sidx, didx, semS, semD, semE, semW = bufs[b]
        eoff = ebase + c * K
        pltpu.make_async_copy(ps_hbm.at[sidx], bufS, semS).wait()
        pltpu.make_async_copy(pd_hbm.at[didx], bufD, semD).wait()
        pltpu.make_async_copy(pe_hbm.at[pl.ds(eoff, K)], bufE, semE).wait()

        def row(r, _):
            for v in range(H // 16):
                ix = (r, pl.ds(v * 16, 16))
                m = bufS[ix] + bufD[ix] + bufE[ix]
                bufS[ix] = jnp.maximum(m, 0.0)
            return 0
        lax.fori_loop(0, K, row, 0)

        pltpu.async_copy(bufS, acc_sh.at[didx], semW, add=True)

    issue(0, 0)

    def body(i, _):
        c0 = 2 * i
        issue(1, c0 + 1)
        finish(0, c0)

        @pl.when(c0 + 2 < NFULL)
        def _():
            issue(0, c0 + 2)
        finish(1, c0 + 1)
        return 0
    lax.fori_loop(0, NFULL // 2, body, 0)

    pltpu.make_async_copy(bufS0, acc_sh.at[didx0], semW0).wait()
    pltpu.make_async_copy(bufS1, acc_sh.at[didx1], semW1).wait()

    toff = ebase + NFULL * K
    pltpu.sync_copy(src_hbm.at[pl.ds(toff, TK)], tsidx)
    pltpu.sync_copy(dst_hbm.at[pl.ds(toff, TK)], tdidx)
    cS = pltpu.async_copy(ps_hbm.at[tsidx], bufS0.at[pl.ds(0, TK)], semS0)
    cD = pltpu.async_copy(pd_hbm.at[tdidx], bufD0.at[pl.ds(0, TK)], semD0)
    cE = pltpu.async_copy(pe_hbm.at[pl.ds(toff, TK)], bufE0.at[pl.ds(0, TK)],
                          semE0)
    cS.wait()
    cD.wait()
    cE.wait()

    def trow(r, _):
        for v in range(H // 16):
            ix = (r, pl.ds(v * 16, 16))
            m = bufS0[ix] + bufD0[ix] + bufE0[ix]
            bufS0[ix] = jnp.maximum(m, 0.0)
        return 0
    lax.fori_loop(0, TK, trow, 0)
    pltpu.sync_copy(bufS0.at[pl.ds(0, TK)], acc_sh.at[tdidx], add=True)

    plsc.subcore_barrier()

    for t in range(NPT // K):
        off = sid * NPT + t * K
        pltpu.sync_copy(acc_sh.at[pl.ds(off, K)], bufS0)
        pltpu.sync_copy(bufS0, acc_out.at[cid, pl.ds(off, K)])
    if rem:
        off = sid * NPT + NPT - rem
        pltpu.sync_copy(acc_sh.at[pl.ds(off, rem)], bufS0.at[pl.ds(0, rem)])
        pltpu.sync_copy(bufS0.at[pl.ds(0, rem)],
                        acc_out.at[cid, pl.ds(off, rem)])


_edge_kernel = pl.kernel(
    _edge_body,
    out_type=[jax.ShapeDtypeStruct((NC, NP, H), _f32)],
    mesh=plsc.VectorSubcoreMesh(
        core_axis_name="c", subcore_axis_name="s",
        num_cores=NC, num_subcores=NS),
    scratch_types=[
        pltpu.VMEM((K, H), _f32),
        pltpu.VMEM((K, H), _f32),
        pltpu.VMEM((K, H), _f32),
        pltpu.VMEM((K,), jnp.int32),
        pltpu.VMEM((K,), jnp.int32),
        pltpu.VMEM((K, H), _f32),
        pltpu.VMEM((K, H), _f32),
        pltpu.VMEM((K, H), _f32),
        pltpu.VMEM((K,), jnp.int32),
        pltpu.VMEM((K,), jnp.int32),
        pltpu.VMEM((TK,), jnp.int32),
        pltpu.VMEM((TK,), jnp.int32),
        pltpu.VMEM_SHARED((NP, H), _f32),
    ] + [pltpu.SemaphoreType.DMA] * 8,
    name="edge",
)


def _deg_body(dst_hbm, deg_out, ones_b, didx, tdidx, zdeg, deg_sh):
    cid = lax.axis_index("c")
    sid = lax.axis_index("s")
    wid = sid * NC + cid
    ebase = wid * EPW

    zero16 = jnp.zeros((16,), _f32)
    one16 = jnp.ones((16,), _f32)

    def zrow(r, _):
        zdeg[pl.ds(r * 16, 16)] = zero16
        return 0
    lax.fori_loop(0, 640 // 16, zrow, 0)
    pltpu.sync_copy(zdeg.at[pl.ds(0, NPT)], deg_sh.at[pl.ds(sid * NPT, NPT)])

    def orow(r, _):
        ones_b[pl.ds(r * 16, 16)] = one16
        return 0
    lax.fori_loop(0, KD // 16, orow, 0)

    plsc.subcore_barrier()

    def chunk(c, _):
        pltpu.sync_copy(dst_hbm.at[pl.ds(ebase + c * KD, KD)], didx)
        pltpu.sync_copy(ones_b, deg_sh.at[didx], add=True)
        return 0
    lax.fori_loop(0, NFULLD, chunk, 0)

    pltpu.sync_copy(dst_hbm.at[pl.ds(ebase + NFULLD * KD, TKD)], tdidx)
    pltpu.sync_copy(ones_b.at[pl.ds(0, TKD)], deg_sh.at[tdidx], add=True)

    plsc.subcore_barrier()

    pltpu.sync_copy(deg_sh.at[pl.ds(sid * NPT, NPT)], zdeg.at[pl.ds(0, NPT)])
    pltpu.sync_copy(zdeg.at[pl.ds(0, NPT)],
                    deg_out.at[pl.ds(cid * NP + sid * NPT, NPT)])


_deg_kernel = pl.kernel(
    _deg_body,
    out_type=[jax.ShapeDtypeStruct((NC * NP,), _f32)],
    mesh=plsc.VectorSubcoreMesh(
        core_axis_name="c", subcore_axis_name="s",
        num_cores=NC, num_subcores=NS),
    scratch_types=[
        pltpu.VMEM((KD,), _f32),
        pltpu.VMEM((KD,), jnp.int32),
        pltpu.VMEM((TKD,), jnp.int32),
        pltpu.VMEM((640,), _f32),
        pltpu.VMEM_SHARED((NP,), _f32),
    ],
    name="deg",
)


def _pe_body(ea_ref, wm3_ref, pe_ref):
    pe_ref[...] = _dot(ea_ref[...], wm3_ref[...])


def _pe_call(edge_attr, Wm3_i):
    return pl.pallas_call(
        _pe_body,
        grid=(E // BE,),
        in_specs=[
            pl.BlockSpec((BE, DE), lambda e: (e, 0)),
            pl.BlockSpec((DE, H), lambda e: (0, 0)),
        ],
        out_specs=pl.BlockSpec((BE, H), lambda e: (e, 0)),
        out_shape=jax.ShapeDtypeStruct((E, H), _f32),
        name="pe",
    )(edge_attr, Wm3_i)


def _proj_body(h_ref, w1_ref, w2_ref, bm_ref, ps_ref, pd_ref):
    h = h_ref[...]
    ps_ref[...] = _dot(h, w1_ref[...])
    pd_ref[...] = _dot(h, w2_ref[...]) + bm_ref[...]


def _proj_call(h, W1, W2, bm_i):
    return pl.pallas_call(
        _proj_body,
        grid=(NBLK,),
        in_specs=[
            pl.BlockSpec((BN, H), lambda i: (i, 0)),
            pl.BlockSpec((H, H), lambda i: (0, 0)),
            pl.BlockSpec((H, H), lambda i: (0, 0)),
            pl.BlockSpec((1, H), lambda i: (0, 0)),
        ],
        out_specs=[
            pl.BlockSpec((BN, H), lambda i: (i, 0)),
            pl.BlockSpec((BN, H), lambda i: (i, 0)),
        ],
        out_shape=[jax.ShapeDtypeStruct((N, H), _f32)] * 2,
        name="proj",
    )(h, W1, W2, bm_i)


def _upd_body(h_ref, ap_ref, deg_ref, wu1_ref, wu2_ref, bu_ref, a_ref,
              h2_ref, sum_ref, ssq_ref):
    i = pl.program_id(0)
    agg = (ap_ref[0] + ap_ref[1]) / deg_ref[...]
    h2 = _dot(h_ref[...], wu1_ref[...]) + _dot(agg, wu2_ref[...]) + bu_ref[...]
    a = a_ref[0, 0]
    h2 = jnp.where(h2 > 0, h2, a * h2)
    h2_ref[...] = h2

    @pl.when(i == 0)
    def _():
        sum_ref[...] = jnp.zeros_like(sum_ref)
        ssq_ref[...] = jnp.zeros_like(ssq_ref)
    sum_ref[...] += jnp.sum(h2, axis=0, keepdims=True)
    ssq_ref[...] += jnp.sum(h2 * h2, axis=0, keepdims=True)


def _upd_call(h, acc_p, deg, Wu1, Wu2, bu_i, a_i):
    return pl.pallas_call(
        _upd_body,
        grid=(NBLK,),
        in_specs=[
            pl.BlockSpec((BN, H), lambda i: (i, 0)),
            pl.BlockSpec((NC, BN, H), lambda i: (0, i, 0)),
            pl.BlockSpec((BN, 1), lambda i: (i, 0)),
            pl.BlockSpec((H, H), lambda i: (0, 0)),
            pl.BlockSpec((H, H), lambda i: (0, 0)),
            pl.BlockSpec((1, H), lambda i: (0, 0)),
            pl.BlockSpec((1, 1), lambda i: (0, 0)),
        ],
        out_specs=[
            pl.BlockSpec((BN, H), lambda i: (i, 0)),
            pl.BlockSpec((1, H), lambda i: (0, 0)),
            pl.BlockSpec((1, H), lambda i: (0, 0)),
        ],
        out_shape=[
            jax.ShapeDtypeStruct((N, H), _f32),
            jax.ShapeDtypeStruct((1, H), _f32),
            jax.ShapeDtypeStruct((1, H), _f32),
        ],
        name="upd",
    )(h, acc_p, deg, Wu1, Wu2, bu_i, a_i)


def _bn_scale_shift(sum_ref, ssq_ref, g_ref, b_ref):
    mu = sum_ref[...] / N
    var = ssq_ref[...] / N - mu * mu
    scale = lax.rsqrt(var + 1e-5) * g_ref[...]
    shift = b_ref[...] - mu * scale
    return scale, shift


def _norm_body(h2_ref, sum_ref, ssq_ref, g_ref, b_ref, w1_ref, w2_ref, bm_ref,
               h_ref, ps_ref, pd_ref):
    scale, shift = _bn_scale_shift(sum_ref, ssq_ref, g_ref, b_ref)
    h = h2_ref[...] * scale + shift
    h_ref[...] = h
    ps_ref[...] = _dot(h, w1_ref[...])
    pd_ref[...] = _dot(h, w2_ref[...]) + bm_ref[...]


def _norm_call(h2, ssum, ssq, gamma_i, beta_i, W1n, W2n, bm_n):
    return pl.pallas_call(
        _norm_body,
        grid=(NBLK,),
        in_specs=[
            pl.BlockSpec((BN, H), lambda i: (i, 0)),
            pl.BlockSpec((1, H), lambda i: (0, 0)),
            pl.BlockSpec((1, H), lambda i: (0, 0)),
            pl.BlockSpec((1, H), lambda i: (0, 0)),
            pl.BlockSpec((1, H), lambda i: (0, 0)),
            pl.BlockSpec((H, H), lambda i: (0, 0)),
            pl.BlockSpec((H, H), lambda i: (0, 0)),
            pl.BlockSpec((1, H), lambda i: (0, 0)),
        ],
        out_specs=[
            pl.BlockSpec((BN, H), lambda i: (i, 0)),
            pl.BlockSpec((BN, H), lambda i: (i, 0)),
            pl.BlockSpec((BN, H), lambda i: (i, 0)),
        ],
        out_shape=[jax.ShapeDtypeStruct((N, H), _f32)] * 3,
        name="norm",
    )(h2, ssum, ssq, gamma_i, beta_i, W1n, W2n, bm_n)


def _final_body(h2_ref, sum_ref, ssq_ref, g_ref, b_ref, batch_ref,
                wh1_ref, bh1_ref, ah_ref, wh2_ref, bh2_ref,
                out_ref, pool_ref, cnt_ref):
    i = pl.program_id(0)
    scale, shift = _bn_scale_shift(sum_ref, ssq_ref, g_ref, b_ref)
    h = h2_ref[...] * scale + shift

    bb = batch_ref[0]
    gids = lax.broadcasted_iota(jnp.int32, (G, BN), 0)
    onehot = (gids == bb).astype(_f32)

    @pl.when(i == 0)
    def _():
        pool_ref[...] = jnp.zeros_like(pool_ref)
        cnt_ref[...] = jnp.zeros_like(cnt_ref)
    pool_ref[...] += _dot(onehot, h)
    cnt_ref[...] += _dot(onehot, jnp.ones_like(h))

    @pl.when(i == NBLK - 1)
    def _():
        pooled = pool_ref[...] / jnp.maximum(cnt_ref[...], 1.0)
        z = _dot(pooled, wh1_ref[...]) + bh1_ref[...]
        ah = ah_ref[0, 0]
        z = jnp.where(z > 0, z, ah * z)
        out_ref[...] = _dot(z, wh2_ref[...]) + bh2_ref[...]


def _final_call(h2, ssum, ssq, gamma_i, beta_i, batch3, Wh1, bh1, ah2, Wh2, bh2):
    return pl.pallas_call(
        _final_body,
        grid=(NBLK,),
        in_specs=[
            pl.BlockSpec((BN, H), lambda i: (i, 0)),
            pl.BlockSpec((1, H), lambda i: (0, 0)),
            pl.BlockSpec((1, H), lambda i: (0, 0)),
            pl.BlockSpec((1, H), lambda i: (0, 0)),
            pl.BlockSpec((1, H), lambda i: (0, 0)),
            pl.BlockSpec((1, 1, BN), lambda i: (i, 0, 0)),
            pl.BlockSpec((H, H), lambda i: (0, 0)),
            pl.BlockSpec((1, H), lambda i: (0, 0)),
            pl.BlockSpec((1, 1), lambda i: (0, 0)),
            pl.BlockSpec((H, 1), lambda i: (0, 0)),
            pl.BlockSpec((1, 1), lambda i: (0, 0)),
        ],
        out_specs=pl.BlockSpec((G, 1), lambda i: (0, 0)),
        out_shape=jax.ShapeDtypeStruct((G, 1), _f32),
        scratch_shapes=[
            pltpu.VMEM((G, H), _f32),
            pltpu.VMEM((G, H), _f32),
        ],
        name="head",
    )(h2, ssum, ssq, gamma_i, beta_i, batch3, Wh1, bh1, ah2, Wh2, bh2)


def kernel(x, edge_index, edge_attr, batch, Wm, bm, Wu, bu, a, gamma, beta,
           Wh1, bh1, ah, Wh2, bh2):
    src = edge_index[0].astype(jnp.int32)
    dst = edge_index[1].astype(jnp.int32)
    batch3 = batch.astype(jnp.int32).reshape(NBLK, 1, BN)

    pe = [_pe_call(edge_attr, Wm[i, 2 * H:]) for i in range(L)]
    (deg_flat,) = _deg_kernel(dst)
    deg = jnp.maximum(deg_flat[0:N] + deg_flat[NP:NP + N], 1.0)[:, None]

    h = x
    ps, pd = _proj_call(h, Wm[0, :H], Wm[0, H:2 * H], bm[0:1])
    for i in range(L):
        (acc_p,) = _edge_kernel(ps, pd, pe[i], src, dst)
        h2, ssum, ssq = _upd_call(
            h, acc_p, deg, Wu[i, :H], Wu[i, H:], bu[i:i + 1],
            a[i].reshape(1, 1))
        if i < L - 1:
            h, ps, pd = _norm_call(
                h2, ssum, ssq, gamma[i:i + 1], beta[i:i + 1],
                Wm[i + 1, :H], Wm[i + 1, H:2 * H], bm[i + 1:i + 2])
        else:
            out = _final_call(
                h2, ssum, ssq, gamma[i:i + 1], beta[i:i + 1], batch3,
                Wh1, bh1.reshape(1, H), jnp.reshape(ah, (1, 1)),
                Wh2, bh2.reshape(1, 1))
    return out


# speedup vs baseline: 5.3198x; 1.1939x over previous
import jax
import jax.numpy as jnp
from jax import lax
from jax.experimental import pallas as pl
from jax.experimental.pallas import tpu as pltpu
from jax.experimental.pallas import tpu_sc as plsc

N = 10000
E = 320000
D = 128
DE = 16
H = 128
L = 3
G = 64

NC = 2
NS = 16
NW = NC * NS
EPW = E // NW
K = 64
NFULL = EPW // K
TK = EPW - NFULL * K
NP = 10112
NPT = NP // NS

KD = 128
NFULLD = EPW // KD
TKD = EPW - NFULLD * KD

BN = 400
NBLK = N // BN
BE = 2000
HP = H // 2

_f32 = jnp.float32
_HIGH = lax.Precision.HIGHEST

def _dot(a, b):
    return jnp.dot(a, b, precision=_HIGH, preferred_element_type=_f32)


def _edge_body(ps_hbm, pd_hbm, pe_hbm, src_hbm, dst_hbm,
               acc_out,
               bufS0, bufD0, bufE0, bufS1, bufD1, bufE1,
               sidx0, sidx1, sidx2, sidx3,
               didx0, didx1, didx2, didx3,
               tsidx, tdidx,
               acc_sh,
               semS0, semD0, semE0, semW0,
               semS1, semD1, semE1, semW1,
               semI0, semI1, semI2, semI3):
    bufs = ((bufS0, semS0, semD0, semE0, semW0),
            (bufS1, semS1, semD1, semE1, semW1))
    bufD = (bufD0, bufD1)
    bufE = (bufE0, bufE1)
    sidxs = (sidx0, sidx1, sidx2, sidx3)
    didxs = (didx0, didx1, didx2, didx3)
    semIs = (semI0, semI1, semI2, semI3)

    cid = lax.axis_index("c")
    sid = lax.axis_index("s")
    wid = sid * NC + cid
    ebase = wid * EPW

    zero16 = jnp.zeros((16,), _f32)

    def zrow(r, _):
        for v in range(H // 16):
            bufS0[r, pl.ds(v * 16, 16)] = zero16
        return 0
    lax.fori_loop(0, K, zrow, 0)

    def acc_slices(sink):
        base = sid * NPT
        for t in range(8):
            sink(base + t * K, K)
        sink(base + 8 * K, 8)

        @pl.when(sid < NS - 1)
        def _():
            sink(base + 520, K)
            sink(base + 584, 48)

    acc_slices(lambda off, n: pltpu.sync_copy(
        bufS0.at[pl.ds(0, n)], acc_sh.at[pl.ds(off, n)]))

    plsc.subcore_barrier()

    def issue_idx(j, c):
        eoff = ebase + c * K
        pltpu.async_copy(src_hbm.at[pl.ds(eoff, K)], sidxs[j], semIs[j])
        pltpu.async_copy(dst_hbm.at[pl.ds(eoff, K)], didxs[j], semIs[j])

    def wait_idx(j, c):
        eoff = ebase + c * K
        pltpu.make_async_copy(src_hbm.at[pl.ds(eoff, K)], sidxs[j],
                              semIs[j]).wait()
        pltpu.make_async_copy(dst_hbm.at[pl.ds(eoff, K)], didxs[j],
                              semIs[j]).wait()

    def gath(b, j, c):
        bufS, semS, semD, semE, semW = bufs[b]
        eoff = ebase + c * K
        pltpu.async_copy(ps_hbm.at[sidxs[j]], bufS, semS)
        pltpu.async_copy(pd_hbm.at[didxs[j]], bufD[b], semD)
        pltpu.async_copy(pe_hbm.at[pl.ds(eoff, K)], bufE[b], semE)

    def drain_scatter(b, j):
        bufS, semS, semD, semE, semW = bufs[b]
        pltpu.make_async_copy(bufS, acc_sh.at[didxs[j]], semW).wait()

    def finish(b, j, c):
        bufS, semS, semD, semE, semW = bufs[b]
        eoff = ebase + c * K
        pltpu.make_async_copy(ps_hbm.at[sidxs[j]], bufS, semS).wait()
        pltpu.make_async_copy(pd_hbm.at[didxs[j]], bufD[b], semD).wait()
        pltpu.make_async_copy(pe_hbm.at[pl.ds(eoff, K)], bufE[b],
                              semE).wait()
        bD = bufD[b]
        bE = bufE[b]

        def row(r, _):
            for v in range(H // 16):
                ix = (r, pl.ds(v * 16, 16))
                m = bufS[ix] + bD[ix] + bE[ix]
                bufS[ix] = jnp.maximum(m, 0.0)
            return 0
        lax.fori_loop(0, K, row, 0)

        pltpu.async_copy(bufS, acc_sh.at[didxs[j]], semW, add=True)

    issue_idx(0, 0)
    issue_idx(1, 1)
    wait_idx(0, 0)
    gath(0, 0, 0)

    def body(i, _):
        c0 = 4 * i
        for k in range(4):
            c = c0 + k

            @pl.when(c + 2 < NFULL)
            def _():
                issue_idx((k + 2) % 4, c + 2)

            @pl.when(c + 1 < NFULL)
            def _():
                @pl.when(c >= 1)
                def _():
                    drain_scatter((k + 1) % 2, (k + 3) % 4)
                wait_idx((k + 1) % 4, c + 1)
                gath((k + 1) % 2, (k + 1) % 4, c + 1)
            finish(k % 2, k % 4, c)
        return 0
    lax.fori_loop(0, NFULL // 4, body, 0)

    drain_scatter((NFULL - 2) % 2, (NFULL - 2) % 4)
    drain_scatter((NFULL - 1) % 2, (NFULL - 1) % 4)

    toff = ebase + NFULL * K
    pltpu.sync_copy(src_hbm.at[pl.ds(toff, TK)], tsidx)
    pltpu.sync_copy(dst_hbm.at[pl.ds(toff, TK)], tdidx)
    cS = pltpu.async_copy(ps_hbm.at[tsidx], bufS0.at[pl.ds(0, TK)], semS0)
    cD = pltpu.async_copy(pd_hbm.at[tdidx], bufD0.at[pl.ds(0, TK)], semD0)
    cE = pltpu.async_copy(pe_hbm.at[pl.ds(toff, TK)], bufE0.at[pl.ds(0, TK)],
                          semE0)
    cS.wait()
    cD.wait()
    cE.wait()

    def trow(r, _):
        for v in range(H // 16):
            ix = (r, pl.ds(v * 16, 16))
            m = bufS0[ix] + bufD0[ix] + bufE0[ix]
            bufS0[ix] = jnp.maximum(m, 0.0)
        return 0
    lax.fori_loop(0, TK, trow, 0)
    pltpu.sync_copy(bufS0.at[pl.ds(0, TK)], acc_sh.at[tdidx], add=True)

    plsc.subcore_barrier()

    def wb(off, n):
        pltpu.sync_copy(acc_sh.at[pl.ds(off, n)], bufS0.at[pl.ds(0, n)])
        pltpu.sync_copy(bufS0.at[pl.ds(0, n)], acc_out.at[cid, pl.ds(off, n)])
    acc_slices(wb)


_edge_kernel = pl.kernel(
    _edge_body,
    out_type=[jax.ShapeDtypeStruct((NC, N, H), _f32)],
    mesh=plsc.VectorSubcoreMesh(
        core_axis_name="c", subcore_axis_name="s",
        num_cores=NC, num_subcores=NS),
    scratch_types=[
        pltpu.VMEM((K, H), _f32),
        pltpu.VMEM((K, H), _f32),
        pltpu.VMEM((K, H), _f32),
        pltpu.VMEM((K, H), _f32),
        pltpu.VMEM((K, H), _f32),
        pltpu.VMEM((K, H), _f32),
    ] + [pltpu.VMEM((K,), jnp.int32)] * 8
    + [
        pltpu.VMEM((TK,), jnp.int32),
        pltpu.VMEM((TK,), jnp.int32),
        pltpu.VMEM_SHARED((N, H), _f32),
    ] + [pltpu.SemaphoreType.DMA] * 12,
    name="edge",
)


def _deg_body(dst_hbm, deg_out, ones_b, didx, tdidx, zdeg, deg_sh):
    cid = lax.axis_index("c")
    sid = lax.axis_index("s")
    wid = sid * NC + cid
    ebase = wid * EPW

    zero16 = jnp.zeros((16,), _f32)
    one16 = jnp.ones((16,), _f32)

    def zrow(r, _):
        zdeg[pl.ds(r * 16, 16)] = zero16
        return 0
    lax.fori_loop(0, 640 // 16, zrow, 0)
    pltpu.sync_copy(zdeg.at[pl.ds(0, NPT)], deg_sh.at[pl.ds(sid * NPT, NPT)])

    def orow(r, _):
        ones_b[pl.ds(r * 16, 16)] = one16
        return 0
    lax.fori_loop(0, KD // 16, orow, 0)

    plsc.subcore_barrier()

    def chunk(c, _):
        pltpu.sync_copy(dst_hbm.at[pl.ds(ebase + c * KD, KD)], didx)
        pltpu.sync_copy(ones_b, deg_sh.at[didx], add=True)
        return 0
    lax.fori_loop(0, NFULLD, chunk, 0)

    pltpu.sync_copy(dst_hbm.at[pl.ds(ebase + NFULLD * KD, TKD)], tdidx)
    pltpu.sync_copy(ones_b.at[pl.ds(0, TKD)], deg_sh.at[tdidx], add=True)

    plsc.subcore_barrier()

    pltpu.sync_copy(deg_sh.at[pl.ds(sid * NPT, NPT)], zdeg.at[pl.ds(0, NPT)])
    pltpu.sync_copy(zdeg.at[pl.ds(0, NPT)],
                    deg_out.at[pl.ds(cid * NP + sid * NPT, NPT)])


_deg_kernel = pl.kernel(
    _deg_body,
    out_type=[jax.ShapeDtypeStruct((NC * NP,), _f32)],
    mesh=plsc.VectorSubcoreMesh(
        core_axis_name="c", subcore_axis_name="s",
        num_cores=NC, num_subcores=NS),
    scratch_types=[
        pltpu.VMEM((KD,), _f32),
        pltpu.VMEM((KD,), jnp.int32),
        pltpu.VMEM((TKD,), jnp.int32),
        pltpu.VMEM((640,), _f32),
        pltpu.VMEM_SHARED((NP,), _f32),
    ],
    name="deg",
)


def _pe_body(ea_ref, wm3_ref, pe_ref):
    pe_ref[...] = _dot(ea_ref[...], wm3_ref[...])


def _pe_call(edge_attr, Wm3_i):
    return pl.pallas_call(
        _pe_body,
        grid=(E // BE,),
        in_specs=[
            pl.BlockSpec((BE, DE), lambda e: (e, 0)),
            pl.BlockSpec((DE, H), lambda e: (0, 0)),
        ],
        out_specs=pl.BlockSpec((BE, H), lambda e: (e, 0)),
        out_shape=jax.ShapeDtypeStruct((E, H), _f32),
        name="pe",
    )(edge_attr, Wm3_i)


def _proj_body(h_ref, w1_ref, w2_ref, bm_ref, ps_ref, pd_ref):
    h = h_ref[...]
    ps_ref[...] = _dot(h, w1_ref[...])
    pd_ref[...] = _dot(h, w2_ref[...]) + bm_ref[...]


def _proj_call(h, W1, W2, bm_i):
    return pl.pallas_call(
        _proj_body,
        grid=(NBLK,),
        in_specs=[
            pl.BlockSpec((BN, H), lambda i: (i, 0)),
            pl.BlockSpec((H, H), lambda i: (0, 0)),
            pl.BlockSpec((H, H), lambda i: (0, 0)),
            pl.BlockSpec((1, H), lambda i: (0, 0)),
        ],
        out_specs=[
            pl.BlockSpec((BN, H), lambda i: (i, 0)),
            pl.BlockSpec((BN, H), lambda i: (i, 0)),
        ],
        out_shape=[jax.ShapeDtypeStruct((N, H), _f32)] * 2,
        name="proj",
    )(h, W1, W2, bm_i)


def _upd_body(h_ref, ap_ref, deg_ref, wu1_ref, wu2_ref, bu_ref, a_ref,
              h2_ref, sum_ref, ssq_ref):
    i = pl.program_id(0)
    agg = (ap_ref[0] + ap_ref[1]) / deg_ref[...]
    h2 = _dot(h_ref[...], wu1_ref[...]) + _dot(agg, wu2_ref[...]) + bu_ref[...]
    a = a_ref[0, 0]
    h2 = jnp.where(h2 > 0, h2, a * h2)
    h2_ref[...] = h2

    @pl.when(i == 0)
    def _():
        sum_ref[...] = jnp.zeros_like(sum_ref)
        ssq_ref[...] = jnp.zeros_like(ssq_ref)
    sum_ref[...] += jnp.sum(h2, axis=0, keepdims=True)
    ssq_ref[...] += jnp.sum(h2 * h2, axis=0, keepdims=True)


def _upd_call(h, acc_p, deg, Wu1, Wu2, bu_i, a_i):
    return pl.pallas_call(
        _upd_body,
        grid=(NBLK,),
        in_specs=[
            pl.BlockSpec((BN, H), lambda i: (i, 0)),
            pl.BlockSpec((NC, BN, H), lambda i: (0, i, 0)),
            pl.BlockSpec((BN, 1), lambda i: (i, 0)),
            pl.BlockSpec((H, H), lambda i: (0, 0)),
            pl.BlockSpec((H, H), lambda i: (0, 0)),
            pl.BlockSpec((1, H), lambda i: (0, 0)),
            pl.BlockSpec((1, 1), lambda i: (0, 0)),
        ],
        out_specs=[
            pl.BlockSpec((BN, H), lambda i: (i, 0)),
            pl.BlockSpec((1, H), lambda i: (0, 0)),
            pl.BlockSpec((1, H), lambda i: (0, 0)),
        ],
        out_shape=[
            jax.ShapeDtypeStruct((N, H), _f32),
            jax.ShapeDtypeStruct((1, H), _f32),
            jax.ShapeDtypeStruct((1, H), _f32),
        ],
        name="upd",
    )(h, acc_p, deg, Wu1, Wu2, bu_i, a_i)


def _bn_scale_shift(sum_ref, ssq_ref, g_ref, b_ref):
    mu = sum_ref[...] / N
    var = ssq_ref[...] / N - mu * mu
    scale = lax.rsqrt(var + 1e-5) * g_ref[...]
    shift = b_ref[...] - mu * scale
    return scale, shift


def _norm_body(h2_ref, sum_ref, ssq_ref, g_ref, b_ref, w1_ref, w2_ref, bm_ref,
               h_ref, ps_ref, pd_ref):
    scale, shift = _bn_scale_shift(sum_ref, ssq_ref, g_ref, b_ref)
    h = h2_ref[...] * scale + shift
    h_ref[...] = h
    ps_ref[...] = _dot(h, w1_ref[...])
    pd_ref[...] = _dot(h, w2_ref[...]) + bm_ref[...]


def _norm_call(h2, ssum, ssq, gamma_i, beta_i, W1n, W2n, bm_n):
    return pl.pallas_call(
        _norm_body,
        grid=(NBLK,),
        in_specs=[
            pl.BlockSpec((BN, H), lambda i: (i, 0)),
            pl.BlockSpec((1, H), lambda i: (0, 0)),
            pl.BlockSpec((1, H), lambda i: (0, 0)),
            pl.BlockSpec((1, H), lambda i: (0, 0)),
            pl.BlockSpec((1, H), lambda i: (0, 0)),
            pl.BlockSpec((H, H), lambda i: (0, 0)),
            pl.BlockSpec((H, H), lambda i: (0, 0)),
            pl.BlockSpec((1, H), lambda i: (0, 0)),
        ],
        out_specs=[
            pl.BlockSpec((BN, H), lambda i: (i, 0)),
            pl.BlockSpec((BN, H), lambda i: (i, 0)),
            pl.BlockSpec((BN, H), lambda i: (i, 0)),
        ],
        out_shape=[jax.ShapeDtypeStruct((N, H), _f32)] * 3,
        name="norm",
    )(h2, ssum, ssq, gamma_i, beta_i, W1n, W2n, bm_n)


def _final_body(h2_ref, sum_ref, ssq_ref, g_ref, b_ref, batch_ref,
                wh1_ref, bh1_ref, ah_ref, wh2_ref, bh2_ref,
                out_ref, pool_ref, cnt_ref):
    i = pl.program_id(0)
    scale, shift = _bn_scale_shift(sum_ref, ssq_ref, g_ref, b_ref)
    h = h2_ref[...] * scale + shift

    bb = batch_ref[0]
    gids = lax.broadcasted_iota(jnp.int32, (G, BN), 0)
    onehot = (gids == bb).astype(_f32)

    @pl.when(i == 0)
    def _():
        pool_ref[...] = jnp.zeros_like(pool_ref)
        cnt_ref[...] = jnp.zeros_like(cnt_ref)
    pool_ref[...] += _dot(onehot, h)
    cnt_ref[...] += _dot(onehot, jnp.ones_like(h))

    @pl.when(i == NBLK - 1)
    def _():
        pooled = pool_ref[...] / jnp.maximum(cnt_ref[...], 1.0)
        z = _dot(pooled, wh1_ref[...]) + bh1_ref[...]
        ah = ah_ref[0, 0]
        z = jnp.where(z > 0, z, ah * z)
        out_ref[...] = _dot(z, wh2_ref[...]) + bh2_ref[...]


def _final_call(h2, ssum, ssq, gamma_i, beta_i, batch3, Wh1, bh1, ah2, Wh2, bh2):
    return pl.pallas_call(
        _final_body,
        grid=(NBLK,),
        in_specs=[
            pl.BlockSpec((BN, H), lambda i: (i, 0)),
            pl.BlockSpec((1, H), lambda i: (0, 0)),
            pl.BlockSpec((1, H), lambda i: (0, 0)),
            pl.BlockSpec((1, H), lambda i: (0, 0)),
            pl.BlockSpec((1, H), lambda i: (0, 0)),
            pl.BlockSpec((1, 1, BN), lambda i: (i, 0, 0)),
            pl.BlockSpec((H, H), lambda i: (0, 0)),
            pl.BlockSpec((1, H), lambda i: (0, 0)),
            pl.BlockSpec((1, 1), lambda i: (0, 0)),
            pl.BlockSpec((H, 1), lambda i: (0, 0)),
            pl.BlockSpec((1, 1), lambda i: (0, 0)),
        ],
        out_specs=pl.BlockSpec((G, 1), lambda i: (0, 0)),
        out_shape=jax.ShapeDtypeStruct((G, 1), _f32),
        scratch_shapes=[
            pltpu.VMEM((G, H), _f32),
            pltpu.VMEM((G, H), _f32),
        ],
        name="head",
    )(h2, ssum, ssq, gamma_i, beta_i, batch3, Wh1, bh1, ah2, Wh2, bh2)


def kernel(x, edge_index, edge_attr, batch, Wm, bm, Wu, bu, a, gamma, beta,
           Wh1, bh1, ah, Wh2, bh2):
    src = edge_index[0].astype(jnp.int32)
    dst = edge_index[1].astype(jnp.int32)
    batch3 = batch.astype(jnp.int32).reshape(NBLK, 1, BN)

    pe = [_pe_call(edge_attr, Wm[i, 2 * H:]) for i in range(L)]
    (deg_flat,) = _deg_kernel(dst)
    deg = jnp.maximum(deg_flat[0:N] + deg_flat[NP:NP + N], 1.0)[:, None]

    h = x
    ps, pd = _proj_call(h, Wm[0, :H], Wm[0, H:2 * H], bm[0:1])
    for i in range(L):
        (acc_p,) = _edge_kernel(ps, pd, pe[i], src, dst)
        h2, ssum, ssq = _upd_call(
            h, acc_p, deg, Wu[i, :H], Wu[i, H:], bu[i:i + 1],
            a[i].reshape(1, 1))
        if i < L - 1:
            h, ps, pd = _norm_call(
                h2, ssum, ssq, gamma[i:i + 1], beta[i:i + 1],
                Wm[i + 1, :H], Wm[i + 1, H:2 * H], bm[i + 1:i + 2])
        else:
            out = _final_call(
                h2, ssum, ssq, gamma[i:i + 1], beta[i:i + 1], batch3,
                Wh1, bh1.reshape(1, H), jnp.reshape(ah, (1, 1)),
                Wh2, bh2.reshape(1, 1))
    return out


# speedup vs baseline: 5.3216x; 1.0003x over previous
import jax
import jax.numpy as jnp
from jax import lax
from jax.experimental import pallas as pl
from jax.experimental.pallas import tpu as pltpu
from jax.experimental.pallas import tpu_sc as plsc

N = 10000
E = 320000
D = 128
DE = 16
H = 128
L = 3
G = 64

NC = 2
NS = 16
NW = NC * NS
EPW = E // NW
K = 64
NFULL = EPW // K
TK = EPW - NFULL * K
NP = 10112
NPT = NP // NS

KD = 128
NFULLD = EPW // KD
TKD = EPW - NFULLD * KD

BN = 400
NBLK = N // BN
BE = 2000

_f32 = jnp.float32
_HIGH = lax.Precision.HIGHEST

def _dot(a, b):
    return jnp.dot(a, b, precision=_HIGH, preferred_element_type=_f32)


def _edge_body(ps_hbm, pd_hbm, pe_hbm, src_hbm, dst_hbm,
               acc_out,
               bufS0, bufD0, bufE0, bufS1, bufD1, bufE1,
               sidx0, sidx1, sidx2, sidx3,
               didx0, didx1, didx2, didx3,
               tsidx, tdidx,
               acc_sh,
               semS0, semD0, semE0, semW0,
               semS1, semD1, semE1, semW1,
               semI0, semI1, semI2, semI3):
    bufs = ((bufS0, semS0, semD0, semE0, semW0),
            (bufS1, semS1, semD1, semE1, semW1))
    bufD = (bufD0, bufD1)
    bufE = (bufE0, bufE1)
    sidxs = (sidx0, sidx1, sidx2, sidx3)
    didxs = (didx0, didx1, didx2, didx3)
    semIs = (semI0, semI1, semI2, semI3)

    cid = lax.axis_index("c")
    sid = lax.axis_index("s")
    wid = sid * NC + cid
    ebase = wid * EPW

    zero16 = jnp.zeros((16,), _f32)

    def zrow(r, _):
        for v in range(H // 16):
            bufS0[r, pl.ds(v * 16, 16)] = zero16
        return 0
    lax.fori_loop(0, K, zrow, 0)

    def acc_slices(sink):
        base = sid * NPT
        for t in range(8):
            sink(base + t * K, K)
        sink(base + 8 * K, 8)

        @pl.when(sid < NS - 1)
        def _():
            sink(base + 520, K)
            sink(base + 584, 48)

    acc_slices(lambda off, n: pltpu.sync_copy(
        bufS0.at[pl.ds(0, n)], acc_sh.at[pl.ds(off, n)]))

    plsc.subcore_barrier()

    def issue_idx(j, c):
        eoff = ebase + c * K
        pltpu.async_copy(src_hbm.at[pl.ds(eoff, K)], sidxs[j], semIs[j])
        pltpu.async_copy(dst_hbm.at[pl.ds(eoff, K)], didxs[j], semIs[j])

    def wait_idx(j, c):
        eoff = ebase + c * K
        pltpu.make_async_copy(src_hbm.at[pl.ds(eoff, K)], sidxs[j],
                              semIs[j]).wait()
        pltpu.make_async_copy(dst_hbm.at[pl.ds(eoff, K)], didxs[j],
                              semIs[j]).wait()

    def gath(b, j, c):
        bufS, semS, semD, semE, semW = bufs[b]
        eoff = ebase + c * K
        pltpu.async_copy(ps_hbm.at[sidxs[j]], bufS, semS)
        pltpu.async_copy(pd_hbm.at[didxs[j]], bufD[b], semD)
        pltpu.async_copy(pe_hbm.at[pl.ds(eoff, K)], bufE[b], semE)

    def drain_scatter(b, j):
        bufS, semS, semD, semE, semW = bufs[b]
        pltpu.make_async_copy(bufS, acc_sh.at[didxs[j]], semW).wait()

    def finish(b, j, c):
        bufS, semS, semD, semE, semW = bufs[b]
        eoff = ebase + c * K
        pltpu.make_async_copy(ps_hbm.at[sidxs[j]], bufS, semS).wait()
        pltpu.make_async_copy(pd_hbm.at[didxs[j]], bufD[b], semD).wait()
        pltpu.make_async_copy(pe_hbm.at[pl.ds(eoff, K)], bufE[b],
                              semE).wait()
        bD = bufD[b]
        bE = bufE[b]

        def row(r, _):
            for v in range(H // 16):
                ix = (r, pl.ds(v * 16, 16))
                m = bufS[ix] + bD[ix] + bE[ix]
                bufS[ix] = jnp.maximum(m, 0.0)
            return 0
        lax.fori_loop(0, K, row, 0)

        pltpu.async_copy(bufS, acc_sh.at[didxs[j]], semW, add=True)

    issue_idx(0, 0)
    issue_idx(1, 1)
    wait_idx(0, 0)
    gath(0, 0, 0)

    def body(i, _):
        c0 = 4 * i
        for k in range(4):
            c = c0 + k

            @pl.when(c + 2 < NFULL)
            def _():
                issue_idx((k + 2) % 4, c + 2)

            @pl.when(c + 1 < NFULL)
            def _():
                @pl.when(c >= 1)
                def _():
                    drain_scatter((k + 1) % 2, (k + 3) % 4)
                wait_idx((k + 1) % 4, c + 1)
                gath((k + 1) % 2, (k + 1) % 4, c + 1)
            finish(k % 2, k % 4, c)
        return 0
    lax.fori_loop(0, NFULL // 4, body, 0)

    drain_scatter((NFULL - 2) % 2, (NFULL - 2) % 4)
    drain_scatter((NFULL - 1) % 2, (NFULL - 1) % 4)

    toff = ebase + NFULL * K
    pltpu.sync_copy(src_hbm.at[pl.ds(toff, TK)], tsidx)
    pltpu.sync_copy(dst_hbm.at[pl.ds(toff, TK)], tdidx)
    cS = pltpu.async_copy(ps_hbm.at[tsidx], bufS0.at[pl.ds(0, TK)], semS0)
    cD = pltpu.async_copy(pd_hbm.at[tdidx], bufD0.at[pl.ds(0, TK)], semD0)
    cE = pltpu.async_copy(pe_hbm.at[pl.ds(toff, TK)], bufE0.at[pl.ds(0, TK)],
                          semE0)
    cS.wait()
    cD.wait()
    cE.wait()

    def trow(r, _):
        for v in range(H // 16):
            ix = (r, pl.ds(v * 16, 16))
            m = bufS0[ix] + bufD0[ix] + bufE0[ix]
            bufS0[ix] = jnp.maximum(m, 0.0)
        return 0
    lax.fori_loop(0, TK, trow, 0)
    pltpu.sync_copy(bufS0.at[pl.ds(0, TK)], acc_sh.at[tdidx], add=True)

    plsc.subcore_barrier()

    def wb(off, n):
        pltpu.sync_copy(acc_sh.at[pl.ds(off, n)], bufS0.at[pl.ds(0, n)])
        pltpu.sync_copy(bufS0.at[pl.ds(0, n)], acc_out.at[cid, pl.ds(off, n)])
    acc_slices(wb)


_edge_kernel = pl.kernel(
    _edge_body,
    out_type=[jax.ShapeDtypeStruct((NC, N, H), _f32)],
    mesh=plsc.VectorSubcoreMesh(
        core_axis_name="c", subcore_axis_name="s",
        num_cores=NC, num_subcores=NS),
    scratch_types=[
        pltpu.VMEM((K, H), _f32),
        pltpu.VMEM((K, H), _f32),
        pltpu.VMEM((K, H), _f32),
        pltpu.VMEM((K, H), _f32),
        pltpu.VMEM((K, H), _f32),
        pltpu.VMEM((K, H), _f32),
    ] + [pltpu.VMEM((K,), jnp.int32)] * 8
    + [
        pltpu.VMEM((TK,), jnp.int32),
        pltpu.VMEM((TK,), jnp.int32),
        pltpu.VMEM_SHARED((N, H), _f32),
    ] + [pltpu.SemaphoreType.DMA] * 12,
    name="edge",
)


def _deg_body(dst_hbm, deg_out, ones_b, didx, tdidx, zdeg, deg_sh):
    cid = lax.axis_index("c")
    sid = lax.axis_index("s")
    wid = sid * NC + cid
    ebase = wid * EPW

    zero16 = jnp.zeros((16,), _f32)
    one16 = jnp.ones((16,), _f32)

    def zrow(r, _):
        zdeg[pl.ds(r * 16, 16)] = zero16
        return 0
    lax.fori_loop(0, 640 // 16, zrow, 0)
    pltpu.sync_copy(zdeg.at[pl.ds(0, NPT)], deg_sh.at[pl.ds(sid * NPT, NPT)])

    def orow(r, _):
        ones_b[pl.ds(r * 16, 16)] = one16
        return 0
    lax.fori_loop(0, KD // 16, orow, 0)

    plsc.subcore_barrier()

    def chunk(c, _):
        pltpu.sync_copy(dst_hbm.at[pl.ds(ebase + c * KD, KD)], didx)
        pltpu.sync_copy(ones_b, deg_sh.at[didx], add=True)
        return 0
    lax.fori_loop(0, NFULLD, chunk, 0)

    pltpu.sync_copy(dst_hbm.at[pl.ds(ebase + NFULLD * KD, TKD)], tdidx)
    pltpu.sync_copy(ones_b.at[pl.ds(0, TKD)], deg_sh.at[tdidx], add=True)

    plsc.subcore_barrier()

    pltpu.sync_copy(deg_sh.at[pl.ds(sid * NPT, NPT)], zdeg.at[pl.ds(0, NPT)])
    pltpu.sync_copy(zdeg.at[pl.ds(0, NPT)],
                    deg_out.at[pl.ds(cid * NP + sid * NPT, NPT)])


_deg_kernel = pl.kernel(
    _deg_body,
    out_type=[jax.ShapeDtypeStruct((NC * NP,), _f32)],
    mesh=plsc.VectorSubcoreMesh(
        core_axis_name="c", subcore_axis_name="s",
        num_cores=NC, num_subcores=NS),
    scratch_types=[
        pltpu.VMEM((KD,), _f32),
        pltpu.VMEM((KD,), jnp.int32),
        pltpu.VMEM((TKD,), jnp.int32),
        pltpu.VMEM((640,), _f32),
        pltpu.VMEM_SHARED((NP,), _f32),
    ],
    name="deg",
)


def _pe_body(ea_ref, wm3_ref, pe_ref):
    pe_ref[...] = _dot(ea_ref[...], wm3_ref[...])


def _pe_call(edge_attr, Wm3_i):
    return pl.pallas_call(
        _pe_body,
        grid=(E // BE,),
        in_specs=[
            pl.BlockSpec((BE, DE), lambda e: (e, 0)),
            pl.BlockSpec((DE, H), lambda e: (0, 0)),
        ],
        out_specs=pl.BlockSpec((BE, H), lambda e: (e, 0)),
        out_shape=jax.ShapeDtypeStruct((E, H), _f32),
        name="pe",
    )(edge_attr, Wm3_i)


def _proj_body(h_ref, w1_ref, w2_ref, bm_ref, ps_ref, pd_ref):
    h = h_ref[...]
    ps_ref[...] = _dot(h, w1_ref[...])
    pd_ref[...] = _dot(h, w2_ref[...]) + bm_ref[...]


def _proj_call(h, W1, W2, bm_i):
    return pl.pallas_call(
        _proj_body,
        grid=(NBLK,),
        in_specs=[
            pl.BlockSpec((BN, H), lambda i: (i, 0)),
            pl.BlockSpec((H, H), lambda i: (0, 0)),
            pl.BlockSpec((H, H), lambda i: (0, 0)),
            pl.BlockSpec((1, H), lambda i: (0, 0)),
        ],
        out_specs=[
            pl.BlockSpec((BN, H), lambda i: (i, 0)),
            pl.BlockSpec((BN, H), lambda i: (i, 0)),
        ],
        out_shape=[jax.ShapeDtypeStruct((N, H), _f32)] * 2,
        name="proj",
    )(h, W1, W2, bm_i)


def _upd_body(h_ref, ap_ref, deg_ref, wu1_ref, wu2_ref, bu_ref, a_ref,
              h2_ref, sum_ref, ssq_ref):
    i = pl.program_id(0)
    agg = (ap_ref[0] + ap_ref[1]) / deg_ref[...]
    h2 = _dot(h_ref[...], wu1_ref[...]) + _dot(agg, wu2_ref[...]) + bu_ref[...]
    a = a_ref[0, 0]
    h2 = jnp.where(h2 > 0, h2, a * h2)
    h2_ref[...] = h2

    @pl.when(i == 0)
    def _():
        sum_ref[...] = jnp.zeros_like(sum_ref)
        ssq_ref[...] = jnp.zeros_like(ssq_ref)
    sum_ref[...] += jnp.sum(h2, axis=0, keepdims=True)
    ssq_ref[...] += jnp.sum(h2 * h2, axis=0, keepdims=True)


def _upd_call(h, acc_p, deg, Wu1, Wu2, bu_i, a_i):
    return pl.pallas_call(
        _upd_body,
        grid=(NBLK,),
        in_specs=[
            pl.BlockSpec((BN, H), lambda i: (i, 0)),
            pl.BlockSpec((NC, BN, H), lambda i: (0, i, 0)),
            pl.BlockSpec((BN, 1), lambda i: (i, 0)),
            pl.BlockSpec((H, H), lambda i: (0, 0)),
            pl.BlockSpec((H, H), lambda i: (0, 0)),
            pl.BlockSpec((1, H), lambda i: (0, 0)),
            pl.BlockSpec((1, 1), lambda i: (0, 0)),
        ],
        out_specs=[
            pl.BlockSpec((BN, H), lambda i: (i, 0)),
            pl.BlockSpec((1, H), lambda i: (0, 0)),
            pl.BlockSpec((1, H), lambda i: (0, 0)),
        ],
        out_shape=[
            jax.ShapeDtypeStruct((N, H), _f32),
            jax.ShapeDtypeStruct((1, H), _f32),
            jax.ShapeDtypeStruct((1, H), _f32),
        ],
        name="upd",
    )(h, acc_p, deg, Wu1, Wu2, bu_i, a_i)


def _bn_scale_shift(sum_ref, ssq_ref, g_ref, b_ref):
    mu = sum_ref[...] / N
    var = ssq_ref[...] / N - mu * mu
    scale = lax.rsqrt(var + 1e-5) * g_ref[...]
    shift = b_ref[...] - mu * scale
    return scale, shift


def _norm_body(h2_ref, sum_ref, ssq_ref, g_ref, b_ref, w1_ref, w2_ref, bm_ref,
               h_ref, ps_ref, pd_ref):
    scale, shift = _bn_scale_shift(sum_ref, ssq_ref, g_ref, b_ref)
    h = h2_ref[...] * scale + shift
    h_ref[...] = h
    ps_ref[...] = _dot(h, w1_ref[...])
    pd_ref[...] = _dot(h, w2_ref[...]) + bm_ref[...]


def _norm_call(h2, ssum, ssq, gamma_i, beta_i, W1n, W2n, bm_n):
    return pl.pallas_call(
        _norm_body,
        grid=(NBLK,),
        in_specs=[
            pl.BlockSpec((BN, H), lambda i: (i, 0)),
            pl.BlockSpec((1, H), lambda i: (0, 0)),
            pl.BlockSpec((1, H), lambda i: (0, 0)),
            pl.BlockSpec((1, H), lambda i: (0, 0)),
            pl.BlockSpec((1, H), lambda i: (0, 0)),
            pl.BlockSpec((H, H), lambda i: (0, 0)),
            pl.BlockSpec((H, H), lambda i: (0, 0)),
            pl.BlockSpec((1, H), lambda i: (0, 0)),
        ],
        out_specs=[
            pl.BlockSpec((BN, H), lambda i: (i, 0)),
            pl.BlockSpec((BN, H), lambda i: (i, 0)),
            pl.BlockSpec((BN, H), lambda i: (i, 0)),
        ],
        out_shape=[jax.ShapeDtypeStruct((N, H), _f32)] * 3,
        name="norm",
    )(h2, ssum, ssq, gamma_i, beta_i, W1n, W2n, bm_n)


def _final_body(h2_ref, sum_ref, ssq_ref, g_ref, b_ref, batch_ref,
                wh1_ref, bh1_ref, ah_ref, wh2_ref, bh2_ref,
                out_ref, pool_ref, cnt_ref):
    i = pl.program_id(0)
    scale, shift = _bn_scale_shift(sum_ref, ssq_ref, g_ref, b_ref)
    h = h2_ref[...] * scale + shift

    bb = batch_ref[0]
    gids = lax.broadcasted_iota(jnp.int32, (G, BN), 0)
    onehot = (gids == bb).astype(_f32)

    @pl.when(i == 0)
    def _():
        pool_ref[...] = jnp.zeros_like(pool_ref)
        cnt_ref[...] = jnp.zeros_like(cnt_ref)
    pool_ref[...] += _dot(onehot, h)
    cnt_ref[...] += _dot(onehot, jnp.ones_like(h))

    @pl.when(i == NBLK - 1)
    def _():
        pooled = pool_ref[...] / jnp.maximum(cnt_ref[...], 1.0)
        z = _dot(pooled, wh1_ref[...]) + bh1_ref[...]
        ah = ah_ref[0, 0]
        z = jnp.where(z > 0, z, ah * z)
        out_ref[...] = _dot(z, wh2_ref[...]) + bh2_ref[...]


def _final_call(h2, ssum, ssq, gamma_i, beta_i, batch3, Wh1, bh1, ah2, Wh2, bh2):
    return pl.pallas_call(
        _final_body,
        grid=(NBLK,),
        in_specs=[
            pl.BlockSpec((BN, H), lambda i: (i, 0)),
            pl.BlockSpec((1, H), lambda i: (0, 0)),
            pl.BlockSpec((1, H), lambda i: (0, 0)),
            pl.BlockSpec((1, H), lambda i: (0, 0)),
            pl.BlockSpec((1, H), lambda i: (0, 0)),
            pl.BlockSpec((1, 1, BN), lambda i: (i, 0, 0)),
            pl.BlockSpec((H, H), lambda i: (0, 0)),
            pl.BlockSpec((1, H), lambda i: (0, 0)),
            pl.BlockSpec((1, 1), lambda i: (0, 0)),
            pl.BlockSpec((H, 1), lambda i: (0, 0)),
            pl.BlockSpec((1, 1), lambda i: (0, 0)),
        ],
        out_specs=pl.BlockSpec((G, 1), lambda i: (0, 0)),
        out_shape=jax.ShapeDtypeStruct((G, 1), _f32),
        scratch_shapes=[
            pltpu.VMEM((G, H), _f32),
            pltpu.VMEM((G, H), _f32),
        ],
        name="head",
    )(h2, ssum, ssq, gamma_i, beta_i, batch3, Wh1, bh1, ah2, Wh2, bh2)


def kernel(x, edge_index, edge_attr, batch, Wm, bm, Wu, bu, a, gamma, beta,
           Wh1, bh1, ah, Wh2, bh2):
    src = edge_index[0].astype(jnp.int32)
    dst = edge_index[1].astype(jnp.int32)
    batch3 = batch.astype(jnp.int32).reshape(NBLK, 1, BN)

    pe = [_pe_call(edge_attr, Wm[i, 2 * H:]) for i in range(L)]
    (deg_flat,) = _deg_kernel(dst)
    deg = jnp.maximum(deg_flat[0:N] + deg_flat[NP:NP + N], 1.0)[:, None]

    h = x
    ps, pd = _proj_call(h, Wm[0, :H], Wm[0, H:2 * H], bm[0:1])
    for i in range(L):
        (acc_p,) = _edge_kernel(ps, pd, pe[i], src, dst)
        h2, ssum, ssq = _upd_call(
            h, acc_p, deg, Wu[i, :H], Wu[i, H:], bu[i:i + 1],
            a[i].reshape(1, 1))
        if i < L - 1:
            h, ps, pd = _norm_call(
                h2, ssum, ssq, gamma[i:i + 1], beta[i:i + 1],
                Wm[i + 1, :H], Wm[i + 1, H:2 * H], bm[i + 1:i + 2])
        else:
            out = _final_call(
                h2, ssum, ssq, gamma[i:i + 1], beta[i:i + 1], batch3,
                Wh1, bh1.reshape(1, H), jnp.reshape(ah, (1, 1)),
                Wh2, bh2.reshape(1, 1))
    return out
